# baseline (speedup 1.0000x reference)
import jax
import jax.numpy as jnp
import numpy as np
from jax.experimental import pallas as pl
from jax.experimental.pallas import tpu as pltpu

_PI = float(np.pi)


def _fps_body(px_ref, py_ref, pz_ref, ox_ref, oy_ref, oz_ref, *, m, s):
    X = px_ref[...]
    Y = py_ref[...]
    Z = pz_ref[...]
    sm = max(m // 128, 1)
    idxg = (jax.lax.broadcasted_iota(jnp.int32, (s, 128), 0) * 128
            + jax.lax.broadcasted_iota(jnp.int32, (s, 128), 1))
    idxm = (jax.lax.broadcasted_iota(jnp.int32, (sm, 128), 0) * 128
            + jax.lax.broadcasted_iota(jnp.int32, (sm, 128), 1))
    qx0 = px_ref[0, 0]
    qy0 = py_ref[0, 0]
    qz0 = pz_ref[0, 0]
    dx = X - qx0
    dy = Y - qy0
    dz = Z - qz0
    dists0 = (dx * dx + dy * dy) + dz * dz
    zf = jnp.zeros((sm, 128), jnp.float32)
    selx0 = jnp.where(idxm == 0, qx0, zf)
    sely0 = jnp.where(idxm == 0, qy0, zf)
    selz0 = jnp.where(idxm == 0, qz0, zf)

    def body(i, c):
        dists, qx, qy, qz, selx, sely, selz = c
        dx = X - qx
        dy = Y - qy
        dz = Z - qz
        d = (dx * dx + dy * dy) + dz * dz
        dists = jnp.minimum(dists, d)
        mx = jnp.max(dists)
        nxt = jnp.min(jnp.where(dists == mx, idxg, jnp.int32(2 ** 30)))
        mask = idxg == nxt
        nqx = jnp.sum(jnp.where(mask, X, 0.0))
        nqy = jnp.sum(jnp.where(mask, Y, 0.0))
        nqz = jnp.sum(jnp.where(mask, Z, 0.0))
        mi = idxm == i
        selx = jnp.where(mi, nqx, selx)
        sely = jnp.where(mi, nqy, sely)
        selz = jnp.where(mi, nqz, selz)
        return (dists, nqx, nqy, nqz, selx, sely, selz)

    c = jax.lax.fori_loop(1, m, body,
                          (dists0, qx0, qy0, qz0, selx0, sely0, selz0))
    ox_ref[...] = c[4]
    oy_ref[...] = c[5]
    oz_ref[...] = c[6]


def _fps(px, py, pz, m):
    s = px.shape[0]
    sm = max(m // 128, 1)
    import functools
    body = functools.partial(_fps_body, m=m, s=s)
    shp = jax.ShapeDtypeStruct((sm, 128), jnp.float32)
    return pl.pallas_call(body, out_shape=(shp, shp, shp))(px, py, pz)


def _bf(v):
    return v.astype(jnp.bfloat16).astype(jnp.float32)


def _qcoord(qref_vals, idxq, g):
    qmask = idxq == g
    return [jnp.sum(jnp.where(qmask, Q, 0.0)) for Q in qref_vals]


def _topk_rounds(C, Xc, Yc, Zc, pmat, qx, qy, qz):
    iota32 = jax.lax.broadcasted_iota(jnp.int32, (1, 32), 1)
    inf = jnp.float32(np.inf)
    ib = jnp.int32(2 ** 30)
    pdxa = jnp.zeros((1, 32), jnp.float32)
    pdya = jnp.zeros((1, 32), jnp.float32)
    pdza = jnp.zeros((1, 32), jnp.float32)
    cola = jnp.zeros((1, 32), jnp.int32)
    for j in range(32):
        m = jnp.min(C)
        pm = jnp.min(jnp.where(C == m, pmat, ib))
        em = pmat == pm
        nx = jnp.sum(jnp.where(em, Xc, 0.0))
        nyv = jnp.sum(jnp.where(em, Yc, 0.0))
        nz = jnp.sum(jnp.where(em, Zc, 0.0))
        jm = iota32 == j
        pdxa = jnp.where(jm, nx - qx, pdxa)
        pdya = jnp.where(jm, nyv - qy, pdya)
        pdza = jnp.where(jm, nz - qz, pdza)
        cola = jnp.where(jm, pm, cola)
        C = jnp.where(em, inf, C)
    return pdxa, pdya, pdza, cola


def _knn1_body(qx_ref, qy_ref, qz_ref, pxt_ref, pyt_ref, pzt_ref,
               pdx_ref, pdy_ref, pdz_ref, col_ref, d_ref, n2_ref,
               *, qper):
    step = pl.program_id(0)
    Xt = pxt_ref[...]
    Yt = pyt_ref[...]
    Zt = pzt_ref[...]

    @pl.when(step == 0)
    def _():
        n2_ref[...] = (Xt * Xt + Yt * Yt) + Zt * Zt

    Xtb, Ytb, Ztb = _bf(Xt), _bf(Yt), _bf(Zt)
    n2t = n2_ref[...]
    idxq = (jax.lax.broadcasted_iota(jnp.int32, (8, 128), 0) * 128
            + jax.lax.broadcasted_iota(jnp.int32, (8, 128), 1))
    QX = qx_ref[...]
    QY = qy_ref[...]
    QZ = qz_ref[...]
    lt = jnp.where(jax.lax.broadcasted_iota(jnp.int32, (128, 128), 0)
                   <= jax.lax.broadcasted_iota(jnp.int32, (128, 128), 1),
                   1.0, 0.0)
    iota64r = jax.lax.broadcasted_iota(jnp.int32, (64, 128), 0).astype(jnp.float32)
    iota_l128f = jax.lax.broadcasted_iota(jnp.int32, (1, 128), 1).astype(jnp.float32)
    iota_l64f = jax.lax.broadcasted_iota(jnp.int32, (1, 64), 1).astype(jnp.float32)
    iota_r128 = jax.lax.broadcasted_iota(jnp.int32, (128, 1), 0)
    hi_p = jax.lax.Precision.HIGHEST

    res = []
    for q in range(qper):
        g = step * qper + q
        qx, qy, qz = _qcoord((QX, QY, QZ), idxq, g)
        ny = (qx * qx + qy * qy) + qz * qz
        D = (ny + n2t) - 2.0 * ((_bf(qx) * Xtb + _bf(qy) * Ytb)
                                + _bf(qz) * Ztb)
        d_ref[q] = D
        rm = jnp.min(D, axis=0, keepdims=True)
        lo = jnp.min(rm)
        hi = jnp.max(rm)
        for _ in range(24):
            mid = 0.5 * (lo + hi)
            cnt = jnp.sum(jnp.where(rm <= mid, 1.0, 0.0))
            sel = cnt < 32.0
            lo = jnp.where(sel, mid, lo)
            hi = jnp.where(sel, hi, mid)
        mask = jnp.where(rm <= hi, 1.0, 0.0)
        cntf = jnp.sum(mask)
        rank = jax.lax.dot_general(mask, lt, (((1,), (0,)), ((), ())),
                                   preferred_element_type=jnp.float32)
        enc = jnp.where(mask > 0, rank - 1.0, jnp.float32(1e9))
        pt = jnp.where(iota64r == enc, 1.0, 0.0)
        Dv = d_ref[q][...]
        cargs = (((1,), (1,)), ((), ()))
        C = jax.lax.dot_general(Dv, pt, cargs, precision=hi_p,
                                preferred_element_type=jnp.float32)
        Xc = jax.lax.dot_general(Xt, pt, cargs, precision=hi_p,
                                 preferred_element_type=jnp.float32)
        Yc = jax.lax.dot_general(Yt, pt, cargs, precision=hi_p,
                                 preferred_element_type=jnp.float32)
        Zc = jax.lax.dot_general(Zt, pt, cargs, precision=hi_p,
                                 preferred_element_type=jnp.float32)
        cval = jax.lax.dot_general(iota_l128f, pt, cargs, precision=hi_p,
                                   preferred_element_type=jnp.float32)
        pmat = cval.astype(jnp.int32) * 128 + iota_r128
        C = jnp.where(iota_l64f < cntf, C, jnp.float32(np.inf))
        res.append(_topk_rounds(C, Xc, Yc, Zc, pmat, qx, qy, qz))

    for q in range(qper):
        pdx_ref[0, q, :] = res[q][0][0, :]
        pdy_ref[0, q, :] = res[q][1][0, :]
        pdz_ref[0, q, :] = res[q][2][0, :]
        col_ref[0, q, :] = res[q][3][0, :]


def _knn1(qx, qy, qz, pxt, pyt, pzt, m, qper=4):
    import functools
    body = functools.partial(_knn1_body, qper=qper)
    shp = jax.ShapeDtypeStruct((m // qper, qper, 32), jnp.float32)
    shpi = jax.ShapeDtypeStruct((m // qper, qper, 32), jnp.int32)
    full_q = pl.BlockSpec((8, 128), lambda i: (0, 0))
    full_p = pl.BlockSpec((128, 128), lambda i: (0, 0))
    outb = pl.BlockSpec((1, qper, 32), lambda i: (i, 0, 0))
    return pl.pallas_call(
        body,
        grid=(m // qper,),
        in_specs=[full_q, full_q, full_q, full_p, full_p, full_p],
        out_specs=[outb, outb, outb, outb],
        out_shape=(shp, shp, shp, shpi),
        scratch_shapes=[pltpu.VMEM((qper, 128, 128), jnp.float32),
                        pltpu.VMEM((128, 128), jnp.float32)],
    )(qx, qy, qz, pxt, pyt, pzt)


def _knn2_body(qx_ref, qy_ref, qz_ref, px_ref, py_ref, pz_ref,
               pdx_ref, pdy_ref, pdz_ref, col_ref, n2_ref, *, qper):
    step = pl.program_id(0)
    X = px_ref[...]
    Y = py_ref[...]
    Z = pz_ref[...]

    @pl.when(step == 0)
    def _():
        n2_ref[...] = (X * X + Y * Y) + Z * Z

    Xb, Yb, Zb = _bf(X), _bf(Y), _bf(Z)
    n2 = n2_ref[...]
    idxq = jax.lax.broadcasted_iota(jnp.int32, (1, 128), 1)
    pmat = (jax.lax.broadcasted_iota(jnp.int32, (8, 128), 0) * 128
            + jax.lax.broadcasted_iota(jnp.int32, (8, 128), 1))
    QX = qx_ref[...]
    QY = qy_ref[...]
    QZ = qz_ref[...]

    res = []
    for q in range(qper):
        g = step * qper + q
        qx, qy, qz = _qcoord((QX, QY, QZ), idxq, g)
        ny = (qx * qx + qy * qy) + qz * qz
        D = (ny + n2) - 2.0 * ((_bf(qx) * Xb + _bf(qy) * Yb) + _bf(qz) * Zb)
        res.append(_topk_rounds(D, X, Y, Z, pmat, qx, qy, qz))

    for q in range(qper):
        pdx_ref[0, q, :] = res[q][0][0, :]
        pdy_ref[0, q, :] = res[q][1][0, :]
        pdz_ref[0, q, :] = res[q][2][0, :]
        col_ref[0, q, :] = res[q][3][0, :]


def _knn2(qx, qy, qz, px, py, pz, m, qper=4):
    import functools
    body = functools.partial(_knn2_body, qper=qper)
    shp = jax.ShapeDtypeStruct((m // qper, qper, 32), jnp.float32)
    shpi = jax.ShapeDtypeStruct((m // qper, qper, 32), jnp.int32)
    full_q = pl.BlockSpec((1, 128), lambda i: (0, 0))
    full_p = pl.BlockSpec((8, 128), lambda i: (0, 0))
    outb = pl.BlockSpec((1, qper, 32), lambda i: (i, 0, 0))
    return pl.pallas_call(
        body,
        grid=(m // qper,),
        in_specs=[full_q, full_q, full_q, full_p, full_p, full_p],
        out_specs=[outb, outb, outb, outb],
        out_shape=(shp, shp, shp, shpi),
        scratch_shapes=[pltpu.VMEM((8, 128), jnp.float32)],
    )(qx, qy, qz, px, py, pz)


def _posenc_feat(pdx, pdy, pdz, n):
    e = jax.lax.broadcasted_iota(jnp.int32, (1, 64), 1)
    k = jnp.maximum(e - 3, 0)
    c = k // 20
    lf = (k % 20) // 2
    is_sin = (k % 2) == 0
    is_coord = e < 3
    coordid = jnp.where(is_coord, e, jnp.minimum(c, 2))
    freq = (jnp.int32(1) << lf).astype(jnp.float32) * _PI
    raw = jnp.where(coordid == 0, pdx,
                    jnp.where(coordid == 1, pdy, pdz))
    scaled = raw * jnp.where(is_coord, jnp.float32(1.0), freq)
    sv = jnp.sin(scaled)
    cv = jnp.cos(scaled)
    return jnp.where(is_coord, raw, jnp.where(is_sin, sv, cv))


def _sa1_body(pdx_ref, pdy_ref, pdz_ref, w0_ref, b0_ref, w1_ref, b1_ref,
              w2_ref, b2_ref, o1_ref):
    n = pdx_ref.shape[0]
    feat = _posenc_feat(pdx_ref[...], pdy_ref[...], pdz_ref[...], n)
    h = jax.lax.dot_general(feat, w0_ref[...], (((1,), (0,)), ((), ())),
                            preferred_element_type=jnp.float32) + b0_ref[...]
    h = jnp.maximum(h, 0.0)
    h = jax.lax.dot_general(h, w1_ref[...], (((1,), (0,)), ((), ())),
                            preferred_element_type=jnp.float32) + b1_ref[...]
    h = jnp.maximum(h, 0.0)
    h = jax.lax.dot_general(h, w2_ref[...], (((1,), (0,)), ((), ())),
                            preferred_element_type=jnp.float32) + b2_ref[...]
    hm = jnp.max(h.reshape(n // 32, 32, 128), axis=1)
    o1_ref[...] = hm


def _sa1_edge(pdxf, pdyf, pdzf, w0p, b0, w1, b1, w2, b2):
    ne = pdxf.shape[0]
    tile = 4096
    grid = ne // tile
    pdb = pl.BlockSpec((tile, 1), lambda i: (i, 0))
    wfull = lambda a: pl.BlockSpec(a.shape, lambda i: (0,) * a.ndim)
    return pl.pallas_call(
        _sa1_body,
        grid=(grid,),
        in_specs=[pdb, pdb, pdb, wfull(w0p), wfull(b0), wfull(w1),
                  wfull(b1), wfull(w2), wfull(b2)],
        out_specs=pl.BlockSpec((tile // 32, 128), lambda i: (i, 0)),
        out_shape=jax.ShapeDtypeStruct((ne // 32, 128), jnp.float32),
    )(pdxf, pdyf, pdzf, w0p, b0, w1, b1, w2, b2)


def _glob_body(x_ref, w0_ref, b0_ref, w1_ref, b1_ref, o_ref):
    h = jax.lax.dot_general(x_ref[...], w0_ref[...], (((1,), (0,)), ((), ())),
                            preferred_element_type=jnp.float32) + b0_ref[...]
    h = jnp.maximum(h, 0.0)
    h = jax.lax.dot_general(h, w1_ref[...], (((1,), (0,)), ((), ())),
                            preferred_element_type=jnp.float32) + b1_ref[...]
    o_ref[...] = h


def _glob(x, w0, b0, w1, b1):
    m = x.shape[0]
    return pl.pallas_call(
        _glob_body,
        out_shape=jax.ShapeDtypeStruct((m, w1.shape[1]), jnp.float32),
    )(x, w0, b0, w1, b1)


def _sa2_body(col_ref, pdx_ref, pdy_ref, pdz_ref, x1_ref,
              w0a_ref, w0b_ref, b0_ref, w1_ref, b1_ref, w2_ref, b2_ref,
              o2_ref):
    n = col_ref.shape[0]
    col = col_ref[...]
    og = jnp.zeros((n, 256), jnp.float32)
    for cb in range(8):
        iota_c = (jax.lax.broadcasted_iota(jnp.int32, (1, 128), 1)
                  + cb * 128)
        ohc = jnp.where(col == iota_c, 1.0, 0.0)
        og = og + jax.lax.dot_general(
            ohc, x1_ref[pl.ds(cb * 128, 128), :], (((1,), (0,)), ((), ())),
            preferred_element_type=jnp.float32)
    feat = _posenc_feat(pdx_ref[...], pdy_ref[...], pdz_ref[...], n)
    h = (jax.lax.dot_general(og, w0a_ref[...], (((1,), (0,)), ((), ())),
                             preferred_element_type=jnp.float32)
         + jax.lax.dot_general(feat, w0b_ref[...], (((1,), (0,)), ((), ())),
                               preferred_element_type=jnp.float32)
         + b0_ref[...])
    h = jnp.maximum(h, 0.0)
    h = jax.lax.dot_general(h, w1_ref[...], (((1,), (0,)), ((), ())),
                            preferred_element_type=jnp.float32) + b1_ref[...]
    h = jnp.maximum(h, 0.0)
    h = jax.lax.dot_general(h, w2_ref[...], (((1,), (0,)), ((), ())),
                            preferred_element_type=jnp.float32) + b2_ref[...]
    hm = jnp.max(h.reshape(n // 32, 32, 512), axis=1)
    o2_ref[...] = hm


def _sa2_edge(colf, pd2xf, pd2yf, pd2zf, x1, w0a, w0bp, b0, w1, b1, w2, b2):
    ne = colf.shape[0]
    tile = 1024
    grid = ne // tile
    cb = pl.BlockSpec((tile, 1), lambda i: (i, 0))
    wfull = lambda a: pl.BlockSpec(a.shape, lambda i: (0,) * a.ndim)
    return pl.pallas_call(
        _sa2_body,
        grid=(grid,),
        in_specs=[cb, cb, cb, cb, wfull(x1), wfull(w0a), wfull(w0bp),
                  wfull(b0), wfull(w1), wfull(b1), wfull(w2), wfull(b2)],
        out_specs=pl.BlockSpec((tile // 32, 512), lambda i: (i, 0)),
        out_shape=jax.ShapeDtypeStruct((ne // 32, 512), jnp.float32),
    )(colf, pd2xf, pd2yf, pd2zf, x1, w0a, w0bp, b0, w1, b1, w2, b2)


def kernel(pos, s1l0w, s1l0b, s1l1w, s1l1b, s1l2w, s1l2b,
           s1g0w, s1g0b, s1g1w, s1g1b,
           s2l0w, s2l0b, s2l1w, s2l1b, s2l2w, s2l2b,
           s2g0w, s2g0b, s2g1w, s2g1b):
    f = jnp.float32
    px = pos[:, 0].reshape(128, 128)
    py = pos[:, 1].reshape(128, 128)
    pz = pos[:, 2].reshape(128, 128)

    p1x, p1y, p1z = _fps(px, py, pz, 1024)
    pdx, pdy, pdz, _ = _knn1(p1x, p1y, p1z, px.T, py.T, pz.T, 1024, qper=4)
    w0p = jnp.pad(s1l0w, ((0, 1), (0, 0)))
    o1 = _sa1_edge(pdx.reshape(32768, 1), pdy.reshape(32768, 1),
                   pdz.reshape(32768, 1), w0p, s1l0b.reshape(1, -1),
                   s1l1w, s1l1b.reshape(1, -1), s1l2w, s1l2b.reshape(1, -1))
    x1 = _glob(o1, s1g0w, s1g0b.reshape(1, -1), s1g1w, s1g1b.reshape(1, -1))

    p2x, p2y, p2z = _fps(p1x, p1y, p1z, 128)
    pd2x, pd2y, pd2z, col2 = _knn2(p2x, p2y, p2z, p1x, p1y, p1z, 128, qper=4)
    w0a = s2l0w[:256]
    w0bp = jnp.pad(s2l0w[256:], ((0, 1), (0, 0)))
    o2 = _sa2_edge(col2.reshape(4096, 1), pd2x.reshape(4096, 1),
                   pd2y.reshape(4096, 1), pd2z.reshape(4096, 1), x1,
                   w0a, w0bp, s2l0b.reshape(1, -1), s2l1w,
                   s2l1b.reshape(1, -1), s2l2w, s2l2b.reshape(1, -1))
    x2 = _glob(o2, s2g0w, s2g0b.reshape(1, -1), s2g1w, s2g1b.reshape(1, -1))

    pos2 = jnp.stack([p2x.reshape(128), p2y.reshape(128),
                      p2z.reshape(128)], axis=1)
    return (x2, pos2)


# speedup vs baseline: 2.7408x; 2.7408x over previous
import jax
import jax.numpy as jnp
import numpy as np
from jax.experimental import pallas as pl
from jax.experimental.pallas import tpu as pltpu

_PI = float(np.pi)


def _fps_body(px_ref, py_ref, pz_ref, ox_ref, oy_ref, oz_ref, *, m, s):
    X = px_ref[...]
    Y = py_ref[...]
    Z = pz_ref[...]
    sm = max(m // 128, 1)
    idxg = (jax.lax.broadcasted_iota(jnp.int32, (s, 128), 0) * 128
            + jax.lax.broadcasted_iota(jnp.int32, (s, 128), 1))
    idxm = (jax.lax.broadcasted_iota(jnp.int32, (sm, 128), 0) * 128
            + jax.lax.broadcasted_iota(jnp.int32, (sm, 128), 1))
    qx0 = px_ref[0, 0]
    qy0 = py_ref[0, 0]
    qz0 = pz_ref[0, 0]
    dx = X - qx0
    dy = Y - qy0
    dz = Z - qz0
    dists0 = (dx * dx + dy * dy) + dz * dz
    zf = jnp.zeros((sm, 128), jnp.float32)
    selx0 = jnp.where(idxm == 0, qx0, zf)
    sely0 = jnp.where(idxm == 0, qy0, zf)
    selz0 = jnp.where(idxm == 0, qz0, zf)

    def body(i, c):
        dists, qx, qy, qz, selx, sely, selz = c
        dx = X - qx
        dy = Y - qy
        dz = Z - qz
        d = (dx * dx + dy * dy) + dz * dz
        dists = jnp.minimum(dists, d)
        mx = jnp.max(dists)
        nxt = jnp.min(jnp.where(dists == mx, idxg, jnp.int32(2 ** 30)))
        mask = idxg == nxt
        nqx = jnp.sum(jnp.where(mask, X, 0.0))
        nqy = jnp.sum(jnp.where(mask, Y, 0.0))
        nqz = jnp.sum(jnp.where(mask, Z, 0.0))
        mi = idxm == i
        selx = jnp.where(mi, nqx, selx)
        sely = jnp.where(mi, nqy, sely)
        selz = jnp.where(mi, nqz, selz)
        return (dists, nqx, nqy, nqz, selx, sely, selz)

    c = jax.lax.fori_loop(1, m, body,
                          (dists0, qx0, qy0, qz0, selx0, sely0, selz0))
    ox_ref[...] = c[4]
    oy_ref[...] = c[5]
    oz_ref[...] = c[6]


def _fps(px, py, pz, m):
    s = px.shape[0]
    sm = max(m // 128, 1)
    import functools
    body = functools.partial(_fps_body, m=m, s=s)
    shp = jax.ShapeDtypeStruct((sm, 128), jnp.float32)
    return pl.pallas_call(body, out_shape=(shp, shp, shp))(px, py, pz)


def _bf(v):
    return v.astype(jnp.bfloat16).astype(jnp.float32)


def _qcoord(qref_vals, idxq, g):
    qmask = idxq == g
    return [jnp.sum(jnp.where(qmask, Q, 0.0)) for Q in qref_vals]


def _topk_rounds_multi(Cs, Xcs, Ycs, Zcs, pmats, qxs, qys, qzs):
    n = len(Cs)
    iota32 = jax.lax.broadcasted_iota(jnp.int32, (1, 32), 1)
    inf = jnp.float32(np.inf)
    pdxa = [jnp.zeros((1, 32), jnp.float32) for _ in range(n)]
    pdya = [jnp.zeros((1, 32), jnp.float32) for _ in range(n)]
    pdza = [jnp.zeros((1, 32), jnp.float32) for _ in range(n)]
    cola = [jnp.zeros((1, 32), jnp.int32) for _ in range(n)]
    Cs = list(Cs)
    ib = jnp.int32(2 ** 30)
    for j in range(32):
        ms = [jnp.min(Cs[q]) for q in range(n)]
        pms = [jnp.min(jnp.where(Cs[q] == ms[q], pmats[q], ib))
               for q in range(n)]
        ems = [pmats[q] == pms[q] for q in range(n)]
        jm = iota32 == j
        for q in range(n):
            em = ems[q]
            nx = jnp.sum(jnp.where(em, Xcs[q], 0.0))
            nyv = jnp.sum(jnp.where(em, Ycs[q], 0.0))
            nz = jnp.sum(jnp.where(em, Zcs[q], 0.0))
            pm = pms[q]
            pdxa[q] = jnp.where(jm, nx - qxs[q], pdxa[q])
            pdya[q] = jnp.where(jm, nyv - qys[q], pdya[q])
            pdza[q] = jnp.where(jm, nz - qzs[q], pdza[q])
            cola[q] = jnp.where(jm, pm, cola[q])
            Cs[q] = jnp.where(em, inf, Cs[q])
    return pdxa, pdya, pdza, cola


def _knn1_body(qx_ref, qy_ref, qz_ref, pxt_ref, pyt_ref, pzt_ref,
               pdx_ref, pdy_ref, pdz_ref, col_ref, d_ref, n2_ref,
               *, qper):
    step = pl.program_id(0)
    Xt = pxt_ref[...]
    Yt = pyt_ref[...]
    Zt = pzt_ref[...]

    @pl.when(step == 0)
    def _():
        n2_ref[...] = (Xt * Xt + Yt * Yt) + Zt * Zt

    Xtb, Ytb, Ztb = _bf(Xt), _bf(Yt), _bf(Zt)
    n2t = n2_ref[...]
    idxq = (jax.lax.broadcasted_iota(jnp.int32, (8, 128), 0) * 128
            + jax.lax.broadcasted_iota(jnp.int32, (8, 128), 1))
    QX = qx_ref[...]
    QY = qy_ref[...]
    QZ = qz_ref[...]
    lt = jnp.where(jax.lax.broadcasted_iota(jnp.int32, (128, 128), 0)
                   <= jax.lax.broadcasted_iota(jnp.int32, (128, 128), 1),
                   1.0, 0.0)
    iota64r = jax.lax.broadcasted_iota(jnp.int32, (64, 128), 0).astype(jnp.float32)
    iota_l128f = jax.lax.broadcasted_iota(jnp.int32, (1, 128), 1).astype(jnp.float32)
    iota_l64f = jax.lax.broadcasted_iota(jnp.int32, (1, 64), 1).astype(jnp.float32)
    iota_r128 = jax.lax.broadcasted_iota(jnp.int32, (128, 1), 0)
    hi_p = jax.lax.Precision.HIGHEST

    qxs, qys, qzs, rms, los, his = [], [], [], [], [], []
    for q in range(qper):
        g = step * qper + q
        qx, qy, qz = _qcoord((QX, QY, QZ), idxq, g)
        qxs.append(qx)
        qys.append(qy)
        qzs.append(qz)
        ny = (qx * qx + qy * qy) + qz * qz
        D = (ny + n2t) - 2.0 * ((_bf(qx) * Xtb + _bf(qy) * Ytb)
                                + _bf(qz) * Ztb)
        d_ref[q] = D
        rm = jnp.min(D, axis=0, keepdims=True)
        rms.append(rm)
        los.append(jnp.min(rm))
        his.append(jnp.max(rm))
    for _ in range(24):
        for q in range(qper):
            mid = 0.5 * (los[q] + his[q])
            cnt = jnp.sum(jnp.where(rms[q] <= mid, 1.0, 0.0))
            sel = cnt < 32.0
            los[q] = jnp.where(sel, mid, los[q])
            his[q] = jnp.where(sel, his[q], mid)
    Cs, Xcs, Ycs, Zcs, pmats = [], [], [], [], []
    cargs = (((1,), (1,)), ((), ()))
    for q in range(qper):
        mask = jnp.where(rms[q] <= his[q], 1.0, 0.0)
        cntf = jnp.sum(mask)
        rank = jax.lax.dot_general(mask, lt, (((1,), (0,)), ((), ())),
                                   preferred_element_type=jnp.float32)
        enc = jnp.where(mask > 0, rank - 1.0, jnp.float32(1e9))
        pt = jnp.where(iota64r == enc, 1.0, 0.0)
        Dv = d_ref[q][...]
        C = jax.lax.dot_general(Dv, pt, cargs, precision=hi_p,
                                preferred_element_type=jnp.float32)
        Xc = jax.lax.dot_general(Xt, pt, cargs, precision=hi_p,
                                 preferred_element_type=jnp.float32)
        Yc = jax.lax.dot_general(Yt, pt, cargs, precision=hi_p,
                                 preferred_element_type=jnp.float32)
        Zc = jax.lax.dot_general(Zt, pt, cargs, precision=hi_p,
                                 preferred_element_type=jnp.float32)
        cval = jax.lax.dot_general(iota_l128f, pt, cargs, precision=hi_p,
                                   preferred_element_type=jnp.float32)
        pmat = cval.astype(jnp.int32) * 128 + iota_r128
        C = jnp.where(iota_l64f < cntf, C, jnp.float32(np.inf))
        Cs.append(C)
        Xcs.append(Xc)
        Ycs.append(Yc)
        Zcs.append(Zc)
        pmats.append(pmat)
    pdxa, pdya, pdza, cola = _topk_rounds_multi(Cs, Xcs, Ycs, Zcs, pmats,
                                                qxs, qys, qzs)
    for q in range(qper):
        pdx_ref[0, q, :] = pdxa[q][0, :]
        pdy_ref[0, q, :] = pdya[q][0, :]
        pdz_ref[0, q, :] = pdza[q][0, :]
        col_ref[0, q, :] = cola[q][0, :]


def _knn1(qx, qy, qz, pxt, pyt, pzt, m, qper=4):
    import functools
    body = functools.partial(_knn1_body, qper=qper)
    shp = jax.ShapeDtypeStruct((m // qper, qper, 32), jnp.float32)
    shpi = jax.ShapeDtypeStruct((m // qper, qper, 32), jnp.int32)
    full_q = pl.BlockSpec((8, 128), lambda i: (0, 0))
    full_p = pl.BlockSpec((128, 128), lambda i: (0, 0))
    outb = pl.BlockSpec((1, qper, 32), lambda i: (i, 0, 0))
    return pl.pallas_call(
        body,
        grid=(m // qper,),
        in_specs=[full_q, full_q, full_q, full_p, full_p, full_p],
        out_specs=[outb, outb, outb, outb],
        out_shape=(shp, shp, shp, shpi),
        scratch_shapes=[pltpu.VMEM((qper, 128, 128), jnp.float32),
                        pltpu.VMEM((128, 128), jnp.float32)],
    )(qx, qy, qz, pxt, pyt, pzt)


def _knn2_body(qx_ref, qy_ref, qz_ref, px_ref, py_ref, pz_ref,
               pdx_ref, pdy_ref, pdz_ref, col_ref, n2_ref, *, qper):
    step = pl.program_id(0)
    X = px_ref[...]
    Y = py_ref[...]
    Z = pz_ref[...]

    @pl.when(step == 0)
    def _():
        n2_ref[...] = (X * X + Y * Y) + Z * Z

    Xb, Yb, Zb = _bf(X), _bf(Y), _bf(Z)
    n2 = n2_ref[...]
    idxq = jax.lax.broadcasted_iota(jnp.int32, (1, 128), 1)
    pmat = (jax.lax.broadcasted_iota(jnp.int32, (8, 128), 0) * 128
            + jax.lax.broadcasted_iota(jnp.int32, (8, 128), 1))
    QX = qx_ref[...]
    QY = qy_ref[...]
    QZ = qz_ref[...]

    qxs, qys, qzs, Ds = [], [], [], []
    for q in range(qper):
        g = step * qper + q
        qx, qy, qz = _qcoord((QX, QY, QZ), idxq, g)
        qxs.append(qx)
        qys.append(qy)
        qzs.append(qz)
        ny = (qx * qx + qy * qy) + qz * qz
        Ds.append((ny + n2) - 2.0 * ((_bf(qx) * Xb + _bf(qy) * Yb)
                                     + _bf(qz) * Zb))
    pdxa, pdya, pdza, cola = _topk_rounds_multi(
        Ds, [X] * qper, [Y] * qper, [Z] * qper, [pmat] * qper,
        qxs, qys, qzs)
    for q in range(qper):
        pdx_ref[0, q, :] = pdxa[q][0, :]
        pdy_ref[0, q, :] = pdya[q][0, :]
        pdz_ref[0, q, :] = pdza[q][0, :]
        col_ref[0, q, :] = cola[q][0, :]


def _knn2(qx, qy, qz, px, py, pz, m, qper=4):
    import functools
    body = functools.partial(_knn2_body, qper=qper)
    shp = jax.ShapeDtypeStruct((m // qper, qper, 32), jnp.float32)
    shpi = jax.ShapeDtypeStruct((m // qper, qper, 32), jnp.int32)
    full_q = pl.BlockSpec((1, 128), lambda i: (0, 0))
    full_p = pl.BlockSpec((8, 128), lambda i: (0, 0))
    outb = pl.BlockSpec((1, qper, 32), lambda i: (i, 0, 0))
    return pl.pallas_call(
        body,
        grid=(m // qper,),
        in_specs=[full_q, full_q, full_q, full_p, full_p, full_p],
        out_specs=[outb, outb, outb, outb],
        out_shape=(shp, shp, shp, shpi),
        scratch_shapes=[pltpu.VMEM((8, 128), jnp.float32)],
    )(qx, qy, qz, px, py, pz)


def _posenc_feat(pdx, pdy, pdz, n):
    e = jax.lax.broadcasted_iota(jnp.int32, (1, 64), 1)
    k = jnp.maximum(e - 3, 0)
    c = k // 20
    lf = (k % 20) // 2
    is_sin = (k % 2) == 0
    is_coord = e < 3
    coordid = jnp.where(is_coord, e, jnp.minimum(c, 2))
    freq = (jnp.int32(1) << lf).astype(jnp.float32) * _PI
    raw = jnp.where(coordid == 0, pdx,
                    jnp.where(coordid == 1, pdy, pdz))
    scaled = raw * jnp.where(is_coord, jnp.float32(1.0), freq)
    sv = jnp.sin(scaled)
    cv = jnp.cos(scaled)
    return jnp.where(is_coord, raw, jnp.where(is_sin, sv, cv))


def _sa1_body(pdx_ref, pdy_ref, pdz_ref, w0_ref, b0_ref, w1_ref, b1_ref,
              w2_ref, b2_ref, o1_ref):
    n = pdx_ref.shape[0]
    feat = _posenc_feat(pdx_ref[...], pdy_ref[...], pdz_ref[...], n)
    h = jax.lax.dot_general(feat, w0_ref[...], (((1,), (0,)), ((), ())),
                            preferred_element_type=jnp.float32) + b0_ref[...]
    h = jnp.maximum(h, 0.0)
    h = jax.lax.dot_general(h, w1_ref[...], (((1,), (0,)), ((), ())),
                            preferred_element_type=jnp.float32) + b1_ref[...]
    h = jnp.maximum(h, 0.0)
    h = jax.lax.dot_general(h, w2_ref[...], (((1,), (0,)), ((), ())),
                            preferred_element_type=jnp.float32) + b2_ref[...]
    hm = jnp.max(h.reshape(n // 32, 32, 128), axis=1)
    o1_ref[...] = hm


def _sa1_edge(pdxf, pdyf, pdzf, w0p, b0, w1, b1, w2, b2):
    ne = pdxf.shape[0]
    tile = 4096
    grid = ne // tile
    pdb = pl.BlockSpec((tile, 1), lambda i: (i, 0))
    wfull = lambda a: pl.BlockSpec(a.shape, lambda i: (0,) * a.ndim)
    return pl.pallas_call(
        _sa1_body,
        grid=(grid,),
        in_specs=[pdb, pdb, pdb, wfull(w0p), wfull(b0), wfull(w1),
                  wfull(b1), wfull(w2), wfull(b2)],
        out_specs=pl.BlockSpec((tile // 32, 128), lambda i: (i, 0)),
        out_shape=jax.ShapeDtypeStruct((ne // 32, 128), jnp.float32),
    )(pdxf, pdyf, pdzf, w0p, b0, w1, b1, w2, b2)


def _glob_body(x_ref, w0_ref, b0_ref, w1_ref, b1_ref, o_ref):
    h = jax.lax.dot_general(x_ref[...], w0_ref[...], (((1,), (0,)), ((), ())),
                            preferred_element_type=jnp.float32) + b0_ref[...]
    h = jnp.maximum(h, 0.0)
    h = jax.lax.dot_general(h, w1_ref[...], (((1,), (0,)), ((), ())),
                            preferred_element_type=jnp.float32) + b1_ref[...]
    o_ref[...] = h


def _glob(x, w0, b0, w1, b1):
    m = x.shape[0]
    return pl.pallas_call(
        _glob_body,
        out_shape=jax.ShapeDtypeStruct((m, w1.shape[1]), jnp.float32),
    )(x, w0, b0, w1, b1)


def _sa2_body(col_ref, pdx_ref, pdy_ref, pdz_ref, x1_ref,
              w0a_ref, w0b_ref, b0_ref, w1_ref, b1_ref, w2_ref, b2_ref,
              o2_ref):
    n = col_ref.shape[0]
    col = col_ref[...]
    og = jnp.zeros((n, 256), jnp.float32)
    for cb in range(8):
        iota_c = (jax.lax.broadcasted_iota(jnp.int32, (1, 128), 1)
                  + cb * 128)
        ohc = jnp.where(col == iota_c, 1.0, 0.0)
        og = og + jax.lax.dot_general(
            ohc, x1_ref[pl.ds(cb * 128, 128), :], (((1,), (0,)), ((), ())),
            preferred_element_type=jnp.float32)
    feat = _posenc_feat(pdx_ref[...], pdy_ref[...], pdz_ref[...], n)
    h = (jax.lax.dot_general(og, w0a_ref[...], (((1,), (0,)), ((), ())),
                             preferred_element_type=jnp.float32)
         + jax.lax.dot_general(feat, w0b_ref[...], (((1,), (0,)), ((), ())),
                               preferred_element_type=jnp.float32)
         + b0_ref[...])
    h = jnp.maximum(h, 0.0)
    h = jax.lax.dot_general(h, w1_ref[...], (((1,), (0,)), ((), ())),
                            preferred_element_type=jnp.float32) + b1_ref[...]
    h = jnp.maximum(h, 0.0)
    h = jax.lax.dot_general(h, w2_ref[...], (((1,), (0,)), ((), ())),
                            preferred_element_type=jnp.float32) + b2_ref[...]
    hm = jnp.max(h.reshape(n // 32, 32, 512), axis=1)
    o2_ref[...] = hm


def _sa2_edge(colf, pd2xf, pd2yf, pd2zf, x1, w0a, w0bp, b0, w1, b1, w2, b2):
    ne = colf.shape[0]
    tile = 1024
    grid = ne // tile
    cb = pl.BlockSpec((tile, 1), lambda i: (i, 0))
    wfull = lambda a: pl.BlockSpec(a.shape, lambda i: (0,) * a.ndim)
    return pl.pallas_call(
        _sa2_body,
        grid=(grid,),
        in_specs=[cb, cb, cb, cb, wfull(x1), wfull(w0a), wfull(w0bp),
                  wfull(b0), wfull(w1), wfull(b1), wfull(w2), wfull(b2)],
        out_specs=pl.BlockSpec((tile // 32, 512), lambda i: (i, 0)),
        out_shape=jax.ShapeDtypeStruct((ne // 32, 512), jnp.float32),
    )(colf, pd2xf, pd2yf, pd2zf, x1, w0a, w0bp, b0, w1, b1, w2, b2)


def kernel(pos, s1l0w, s1l0b, s1l1w, s1l1b, s1l2w, s1l2b,
           s1g0w, s1g0b, s1g1w, s1g1b,
           s2l0w, s2l0b, s2l1w, s2l1b, s2l2w, s2l2b,
           s2g0w, s2g0b, s2g1w, s2g1b):
    f = jnp.float32
    px = pos[:, 0].reshape(128, 128)
    py = pos[:, 1].reshape(128, 128)
    pz = pos[:, 2].reshape(128, 128)

    p1x, p1y, p1z = _fps(px, py, pz, 1024)
    pdx, pdy, pdz, _ = _knn1(p1x, p1y, p1z, px.T, py.T, pz.T, 1024, qper=4)
    w0p = jnp.pad(s1l0w, ((0, 1), (0, 0)))
    o1 = _sa1_edge(pdx.reshape(32768, 1), pdy.reshape(32768, 1),
                   pdz.reshape(32768, 1), w0p, s1l0b.reshape(1, -1),
                   s1l1w, s1l1b.reshape(1, -1), s1l2w, s1l2b.reshape(1, -1))
    x1 = _glob(o1, s1g0w, s1g0b.reshape(1, -1), s1g1w, s1g1b.reshape(1, -1))

    p2x, p2y, p2z = _fps(p1x, p1y, p1z, 128)
    pd2x, pd2y, pd2z, col2 = _knn2(p2x, p2y, p2z, p1x, p1y, p1z, 128, qper=4)
    w0a = s2l0w[:256]
    w0bp = jnp.pad(s2l0w[256:], ((0, 1), (0, 0)))
    o2 = _sa2_edge(col2.reshape(4096, 1), pd2x.reshape(4096, 1),
                   pd2y.reshape(4096, 1), pd2z.reshape(4096, 1), x1,
                   w0a, w0bp, s2l0b.reshape(1, -1), s2l1w,
                   s2l1b.reshape(1, -1), s2l2w, s2l2b.reshape(1, -1))
    x2 = _glob(o2, s2g0w, s2g0b.reshape(1, -1), s2g1w, s2g1b.reshape(1, -1))

    pos2 = jnp.stack([p2x.reshape(128), p2y.reshape(128),
                      p2z.reshape(128)], axis=1)
    return (x2, pos2)


# speedup vs baseline: 3.8560x; 1.4069x over previous
import jax
import jax.numpy as jnp
import numpy as np
from jax.experimental import pallas as pl
from jax.experimental.pallas import tpu as pltpu

_PI = float(np.pi)


def _fps_body(px_ref, py_ref, pz_ref, ox_ref, oy_ref, oz_ref, *, m, s):
    X = px_ref[...]
    Y = py_ref[...]
    Z = pz_ref[...]
    sm = max(m // 128, 1)
    idxg = (jax.lax.broadcasted_iota(jnp.int32, (s, 128), 0) * 128
            + jax.lax.broadcasted_iota(jnp.int32, (s, 128), 1))
    idxm = (jax.lax.broadcasted_iota(jnp.int32, (sm, 128), 0) * 128
            + jax.lax.broadcasted_iota(jnp.int32, (sm, 128), 1))
    qx0 = px_ref[0, 0]
    qy0 = py_ref[0, 0]
    qz0 = pz_ref[0, 0]
    dx = X - qx0
    dy = Y - qy0
    dz = Z - qz0
    dists0 = (dx * dx + dy * dy) + dz * dz
    zf = jnp.zeros((sm, 128), jnp.float32)
    selx0 = jnp.where(idxm == 0, qx0, zf)
    sely0 = jnp.where(idxm == 0, qy0, zf)
    selz0 = jnp.where(idxm == 0, qz0, zf)

    def body(i, c):
        dists, qx, qy, qz, selx, sely, selz = c
        dx = X - qx
        dy = Y - qy
        dz = Z - qz
        d = (dx * dx + dy * dy) + dz * dz
        dists = jnp.minimum(dists, d)
        mx = jnp.max(dists)
        nxt = jnp.min(jnp.where(dists == mx, idxg, jnp.int32(2 ** 30)))
        mask = idxg == nxt
        nqx = jnp.sum(jnp.where(mask, X, 0.0))
        nqy = jnp.sum(jnp.where(mask, Y, 0.0))
        nqz = jnp.sum(jnp.where(mask, Z, 0.0))
        mi = idxm == i
        selx = jnp.where(mi, nqx, selx)
        sely = jnp.where(mi, nqy, sely)
        selz = jnp.where(mi, nqz, selz)
        return (dists, nqx, nqy, nqz, selx, sely, selz)

    c = jax.lax.fori_loop(1, m, body,
                          (dists0, qx0, qy0, qz0, selx0, sely0, selz0))
    ox_ref[...] = c[4]
    oy_ref[...] = c[5]
    oz_ref[...] = c[6]


def _fps(px, py, pz, m):
    s = px.shape[0]
    sm = max(m // 128, 1)
    import functools
    body = functools.partial(_fps_body, m=m, s=s)
    shp = jax.ShapeDtypeStruct((sm, 128), jnp.float32)
    return pl.pallas_call(body, out_shape=(shp, shp, shp))(px, py, pz)


def _bf(v):
    return v.astype(jnp.bfloat16).astype(jnp.float32)


def _qcoord(qref_vals, idxq, g):
    qmask = idxq == g
    return [jnp.sum(jnp.where(qmask, Q, 0.0)) for Q in qref_vals]


def _topk_rounds_multi(Cs, Xcs, Ycs, Zcs, pmats, qxs, qys, qzs):
    n = len(Cs)
    iota32 = jax.lax.broadcasted_iota(jnp.int32, (1, 32), 1)
    inf = jnp.float32(np.inf)
    pdxa = [jnp.zeros((1, 32), jnp.float32) for _ in range(n)]
    pdya = [jnp.zeros((1, 32), jnp.float32) for _ in range(n)]
    pdza = [jnp.zeros((1, 32), jnp.float32) for _ in range(n)]
    cola = [jnp.zeros((1, 32), jnp.int32) for _ in range(n)]
    Cs = list(Cs)
    ib = jnp.int32(2 ** 30)
    for j in range(32):
        ms = [jnp.min(Cs[q]) for q in range(n)]
        pms = [jnp.min(jnp.where(Cs[q] == ms[q], pmats[q], ib))
               for q in range(n)]
        ems = [pmats[q] == pms[q] for q in range(n)]
        jm = iota32 == j
        for q in range(n):
            em = ems[q]
            nx = jnp.sum(jnp.where(em, Xcs[q], 0.0))
            nyv = jnp.sum(jnp.where(em, Ycs[q], 0.0))
            nz = jnp.sum(jnp.where(em, Zcs[q], 0.0))
            pm = pms[q]
            pdxa[q] = jnp.where(jm, nx - qxs[q], pdxa[q])
            pdya[q] = jnp.where(jm, nyv - qys[q], pdya[q])
            pdza[q] = jnp.where(jm, nz - qzs[q], pdza[q])
            cola[q] = jnp.where(jm, pm, cola[q])
            Cs[q] = jnp.where(em, inf, Cs[q])
    return pdxa, pdya, pdza, cola


def _knn1_body(qx_ref, qy_ref, qz_ref, pxt_ref, pyt_ref, pzt_ref,
               pdx_ref, pdy_ref, pdz_ref, col_ref, d_ref, n2_ref,
               *, qper):
    step = pl.program_id(0)
    Xt = pxt_ref[...]
    Yt = pyt_ref[...]
    Zt = pzt_ref[...]

    @pl.when(step == 0)
    def _():
        n2_ref[...] = (Xt * Xt + Yt * Yt) + Zt * Zt

    Xtb, Ytb, Ztb = _bf(Xt), _bf(Yt), _bf(Zt)
    n2t = n2_ref[...]
    idxq = (jax.lax.broadcasted_iota(jnp.int32, (8, 128), 0) * 128
            + jax.lax.broadcasted_iota(jnp.int32, (8, 128), 1))
    QX = qx_ref[...]
    QY = qy_ref[...]
    QZ = qz_ref[...]
    lt = jnp.where(jax.lax.broadcasted_iota(jnp.int32, (128, 128), 0)
                   <= jax.lax.broadcasted_iota(jnp.int32, (128, 128), 1),
                   1.0, 0.0)
    iota64r = jax.lax.broadcasted_iota(jnp.int32, (64, 128), 0).astype(jnp.float32)
    iota_l128f = jax.lax.broadcasted_iota(jnp.int32, (1, 128), 1).astype(jnp.float32)
    iota_l64f = jax.lax.broadcasted_iota(jnp.int32, (1, 64), 1).astype(jnp.float32)
    iota_r128 = jax.lax.broadcasted_iota(jnp.int32, (128, 1), 0)
    hi_p = jax.lax.Precision.HIGHEST

    qxs, qys, qzs, rms, los, his = [], [], [], [], [], []
    for q in range(qper):
        g = step * qper + q
        qx, qy, qz = _qcoord((QX, QY, QZ), idxq, g)
        qxs.append(qx)
        qys.append(qy)
        qzs.append(qz)
        ny = (qx * qx + qy * qy) + qz * qz
        D = (ny + n2t) - 2.0 * ((_bf(qx) * Xtb + _bf(qy) * Ytb)
                                + _bf(qz) * Ztb)
        d_ref[q] = D
        rm = jnp.min(D, axis=0, keepdims=True)
        rms.append(rm)
        los.append(jnp.min(rm))
        his.append(jnp.max(rm))
    for _ in range(24):
        for q in range(qper):
            mid = 0.5 * (los[q] + his[q])
            cnt = jnp.sum(jnp.where(rms[q] <= mid, 1.0, 0.0))
            sel = cnt < 32.0
            los[q] = jnp.where(sel, mid, los[q])
            his[q] = jnp.where(sel, his[q], mid)
    Cs, Xcs, Ycs, Zcs, pmats = [], [], [], [], []
    cargs = (((1,), (1,)), ((), ()))
    for q in range(qper):
        mask = jnp.where(rms[q] <= his[q], 1.0, 0.0)
        cntf = jnp.sum(mask)
        rank = jax.lax.dot_general(mask, lt, (((1,), (0,)), ((), ())),
                                   preferred_element_type=jnp.float32)
        enc = jnp.where(mask > 0, rank - 1.0, jnp.float32(1e9))
        pt = jnp.where(iota64r == enc, 1.0, 0.0)
        Dv = d_ref[q][...]
        C = jax.lax.dot_general(Dv, pt, cargs, precision=hi_p,
                                preferred_element_type=jnp.float32)
        Xc = jax.lax.dot_general(Xt, pt, cargs, precision=hi_p,
                                 preferred_element_type=jnp.float32)
        Yc = jax.lax.dot_general(Yt, pt, cargs, precision=hi_p,
                                 preferred_element_type=jnp.float32)
        Zc = jax.lax.dot_general(Zt, pt, cargs, precision=hi_p,
                                 preferred_element_type=jnp.float32)
        cval = jax.lax.dot_general(iota_l128f, pt, cargs, precision=hi_p,
                                   preferred_element_type=jnp.float32)
        pmat = cval.astype(jnp.int32) * 128 + iota_r128
        C = jnp.where(iota_l64f < cntf, C, jnp.float32(np.inf))
        Cs.append(C)
        Xcs.append(Xc)
        Ycs.append(Yc)
        Zcs.append(Zc)
        pmats.append(pmat)
    pdxa, pdya, pdza, cola = _topk_rounds_multi(Cs, Xcs, Ycs, Zcs, pmats,
                                                qxs, qys, qzs)
    for q in range(qper):
        pdx_ref[0, q, :] = pdxa[q][0, :]
        pdy_ref[0, q, :] = pdya[q][0, :]
        pdz_ref[0, q, :] = pdza[q][0, :]
        col_ref[0, q, :] = cola[q][0, :]


def _knn1(qx, qy, qz, pxt, pyt, pzt, m, qper=4):
    import functools
    body = functools.partial(_knn1_body, qper=qper)
    shp = jax.ShapeDtypeStruct((m // qper, qper, 32), jnp.float32)
    shpi = jax.ShapeDtypeStruct((m // qper, qper, 32), jnp.int32)
    full_q = pl.BlockSpec((8, 128), lambda i: (0, 0))
    full_p = pl.BlockSpec((128, 128), lambda i: (0, 0))
    outb = pl.BlockSpec((1, qper, 32), lambda i: (i, 0, 0))
    return pl.pallas_call(
        body,
        grid=(m // qper,),
        in_specs=[full_q, full_q, full_q, full_p, full_p, full_p],
        out_specs=[outb, outb, outb, outb],
        out_shape=(shp, shp, shp, shpi),
        scratch_shapes=[pltpu.VMEM((qper, 128, 128), jnp.float32),
                        pltpu.VMEM((128, 128), jnp.float32)],
    )(qx, qy, qz, pxt, pyt, pzt)


def _knn2_body(qx_ref, qy_ref, qz_ref, px_ref, py_ref, pz_ref,
               pdx_ref, pdy_ref, pdz_ref, col_ref, n2_ref, *, qper):
    step = pl.program_id(0)
    X = px_ref[...]
    Y = py_ref[...]
    Z = pz_ref[...]

    @pl.when(step == 0)
    def _():
        n2_ref[...] = (X * X + Y * Y) + Z * Z

    Xb, Yb, Zb = _bf(X), _bf(Y), _bf(Z)
    n2 = n2_ref[...]
    idxq = jax.lax.broadcasted_iota(jnp.int32, (1, 128), 1)
    pmat = (jax.lax.broadcasted_iota(jnp.int32, (8, 128), 0) * 128
            + jax.lax.broadcasted_iota(jnp.int32, (8, 128), 1))
    QX = qx_ref[...]
    QY = qy_ref[...]
    QZ = qz_ref[...]

    qxs, qys, qzs, Ds = [], [], [], []
    for q in range(qper):
        g = step * qper + q
        qx, qy, qz = _qcoord((QX, QY, QZ), idxq, g)
        qxs.append(qx)
        qys.append(qy)
        qzs.append(qz)
        ny = (qx * qx + qy * qy) + qz * qz
        Ds.append((ny + n2) - 2.0 * ((_bf(qx) * Xb + _bf(qy) * Yb)
                                     + _bf(qz) * Zb))
    pdxa, pdya, pdza, cola = _topk_rounds_multi(
        Ds, [X] * qper, [Y] * qper, [Z] * qper, [pmat] * qper,
        qxs, qys, qzs)
    for q in range(qper):
        pdx_ref[0, q, :] = pdxa[q][0, :]
        pdy_ref[0, q, :] = pdya[q][0, :]
        pdz_ref[0, q, :] = pdza[q][0, :]
        col_ref[0, q, :] = cola[q][0, :]


def _knn2(qx, qy, qz, px, py, pz, m, qper=4):
    import functools
    body = functools.partial(_knn2_body, qper=qper)
    shp = jax.ShapeDtypeStruct((m // qper, qper, 32), jnp.float32)
    shpi = jax.ShapeDtypeStruct((m // qper, qper, 32), jnp.int32)
    full_q = pl.BlockSpec((1, 128), lambda i: (0, 0))
    full_p = pl.BlockSpec((8, 128), lambda i: (0, 0))
    outb = pl.BlockSpec((1, qper, 32), lambda i: (i, 0, 0))
    return pl.pallas_call(
        body,
        grid=(m // qper,),
        in_specs=[full_q, full_q, full_q, full_p, full_p, full_p],
        out_specs=[outb, outb, outb, outb],
        out_shape=(shp, shp, shp, shpi),
        scratch_shapes=[pltpu.VMEM((8, 128), jnp.float32)],
    )(qx, qy, qz, px, py, pz)


def _posenc_feat(pdx, pdy, pdz, n):
    e = jax.lax.broadcasted_iota(jnp.int32, (1, 64), 1)
    k = jnp.maximum(e - 3, 0)
    c = k // 20
    lf = (k % 20) // 2
    is_sin = (k % 2) == 0
    is_coord = e < 3
    coordid = jnp.where(is_coord, e, jnp.minimum(c, 2))
    freq = (jnp.int32(1) << lf).astype(jnp.float32) * _PI
    raw = jnp.where(coordid == 0, pdx,
                    jnp.where(coordid == 1, pdy, pdz))
    scaled = raw * jnp.where(is_coord, jnp.float32(1.0), freq)
    sv = jnp.sin(scaled)
    cv = jnp.cos(scaled)
    return jnp.where(is_coord, raw, jnp.where(is_sin, sv, cv))


def _sa1_body(pdx_ref, pdy_ref, pdz_ref, w0_ref, b0_ref, w1_ref, b1_ref,
              w2_ref, b2_ref, o1_ref):
    n = pdx_ref.shape[0]
    feat = _posenc_feat(pdx_ref[...], pdy_ref[...], pdz_ref[...], n)
    h = jax.lax.dot_general(feat, w0_ref[...], (((1,), (0,)), ((), ())),
                            preferred_element_type=jnp.float32) + b0_ref[...]
    h = jnp.maximum(h, 0.0)
    h = jax.lax.dot_general(h, w1_ref[...], (((1,), (0,)), ((), ())),
                            preferred_element_type=jnp.float32) + b1_ref[...]
    h = jnp.maximum(h, 0.0)
    h = jax.lax.dot_general(h, w2_ref[...], (((1,), (0,)), ((), ())),
                            preferred_element_type=jnp.float32) + b2_ref[...]
    hm = jnp.max(h.reshape(n // 32, 32, 128), axis=1)
    o1_ref[...] = hm


def _sa1_edge(pdxf, pdyf, pdzf, w0p, b0, w1, b1, w2, b2):
    ne = pdxf.shape[0]
    tile = 4096
    grid = ne // tile
    pdb = pl.BlockSpec((tile, 1), lambda i: (i, 0))
    wfull = lambda a: pl.BlockSpec(a.shape, lambda i: (0,) * a.ndim)
    return pl.pallas_call(
        _sa1_body,
        grid=(grid,),
        in_specs=[pdb, pdb, pdb, wfull(w0p), wfull(b0), wfull(w1),
                  wfull(b1), wfull(w2), wfull(b2)],
        out_specs=pl.BlockSpec((tile // 32, 128), lambda i: (i, 0)),
        out_shape=jax.ShapeDtypeStruct((ne // 32, 128), jnp.float32),
    )(pdxf, pdyf, pdzf, w0p, b0, w1, b1, w2, b2)


def _glob_body(x_ref, w0_ref, b0_ref, w1_ref, b1_ref, o_ref):
    h = jax.lax.dot_general(x_ref[...], w0_ref[...], (((1,), (0,)), ((), ())),
                            preferred_element_type=jnp.float32) + b0_ref[...]
    h = jnp.maximum(h, 0.0)
    h = jax.lax.dot_general(h, w1_ref[...], (((1,), (0,)), ((), ())),
                            preferred_element_type=jnp.float32) + b1_ref[...]
    o_ref[...] = h


def _glob(x, w0, b0, w1, b1):
    m = x.shape[0]
    return pl.pallas_call(
        _glob_body,
        out_shape=jax.ShapeDtypeStruct((m, w1.shape[1]), jnp.float32),
    )(x, w0, b0, w1, b1)


def _sa2_body(col_ref, pdx_ref, pdy_ref, pdz_ref, x1_ref,
              w0a_ref, w0b_ref, b0_ref, w1_ref, b1_ref, w2_ref, b2_ref,
              o2_ref):
    n = col_ref.shape[0]
    col = col_ref[...]
    og = jnp.zeros((n, 256), jnp.float32)
    for cb in range(8):
        iota_c = (jax.lax.broadcasted_iota(jnp.int32, (1, 128), 1)
                  + cb * 128)
        ohc = jnp.where(col == iota_c, 1.0, 0.0)
        og = og + jax.lax.dot_general(
            ohc, x1_ref[pl.ds(cb * 128, 128), :], (((1,), (0,)), ((), ())),
            preferred_element_type=jnp.float32)
    feat = _posenc_feat(pdx_ref[...], pdy_ref[...], pdz_ref[...], n)
    h = (jax.lax.dot_general(og, w0a_ref[...], (((1,), (0,)), ((), ())),
                             preferred_element_type=jnp.float32)
         + jax.lax.dot_general(feat, w0b_ref[...], (((1,), (0,)), ((), ())),
                               preferred_element_type=jnp.float32)
         + b0_ref[...])
    h = jnp.maximum(h, 0.0)
    h = jax.lax.dot_general(h, w1_ref[...], (((1,), (0,)), ((), ())),
                            preferred_element_type=jnp.float32) + b1_ref[...]
    h = jnp.maximum(h, 0.0)
    h = jax.lax.dot_general(h, w2_ref[...], (((1,), (0,)), ((), ())),
                            preferred_element_type=jnp.float32) + b2_ref[...]
    hm = jnp.max(h.reshape(n // 32, 32, 512), axis=1)
    o2_ref[...] = hm


def _sa2_edge(colf, pd2xf, pd2yf, pd2zf, x1, w0a, w0bp, b0, w1, b1, w2, b2):
    ne = colf.shape[0]
    tile = 1024
    grid = ne // tile
    cb = pl.BlockSpec((tile, 1), lambda i: (i, 0))
    wfull = lambda a: pl.BlockSpec(a.shape, lambda i: (0,) * a.ndim)
    return pl.pallas_call(
        _sa2_body,
        grid=(grid,),
        in_specs=[cb, cb, cb, cb, wfull(x1), wfull(w0a), wfull(w0bp),
                  wfull(b0), wfull(w1), wfull(b1), wfull(w2), wfull(b2)],
        out_specs=pl.BlockSpec((tile // 32, 512), lambda i: (i, 0)),
        out_shape=jax.ShapeDtypeStruct((ne // 32, 512), jnp.float32),
    )(colf, pd2xf, pd2yf, pd2zf, x1, w0a, w0bp, b0, w1, b1, w2, b2)


def kernel(pos, s1l0w, s1l0b, s1l1w, s1l1b, s1l2w, s1l2b,
           s1g0w, s1g0b, s1g1w, s1g1b,
           s2l0w, s2l0b, s2l1w, s2l1b, s2l2w, s2l2b,
           s2g0w, s2g0b, s2g1w, s2g1b):
    f = jnp.float32
    px = pos[:, 0].reshape(128, 128)
    py = pos[:, 1].reshape(128, 128)
    pz = pos[:, 2].reshape(128, 128)

    p1x, p1y, p1z = _fps(px, py, pz, 1024)
    pdx, pdy, pdz, _ = _knn1(p1x, p1y, p1z, px.T, py.T, pz.T, 1024, qper=8)
    w0p = jnp.pad(s1l0w, ((0, 1), (0, 0)))
    o1 = _sa1_edge(pdx.reshape(32768, 1), pdy.reshape(32768, 1),
                   pdz.reshape(32768, 1), w0p, s1l0b.reshape(1, -1),
                   s1l1w, s1l1b.reshape(1, -1), s1l2w, s1l2b.reshape(1, -1))
    x1 = _glob(o1, s1g0w, s1g0b.reshape(1, -1), s1g1w, s1g1b.reshape(1, -1))

    p2x, p2y, p2z = _fps(p1x, p1y, p1z, 128)
    pd2x, pd2y, pd2z, col2 = _knn2(p2x, p2y, p2z, p1x, p1y, p1z, 128, qper=8)
    w0a = s2l0w[:256]
    w0bp = jnp.pad(s2l0w[256:], ((0, 1), (0, 0)))
    o2 = _sa2_edge(col2.reshape(4096, 1), pd2x.reshape(4096, 1),
                   pd2y.reshape(4096, 1), pd2z.reshape(4096, 1), x1,
                   w0a, w0bp, s2l0b.reshape(1, -1), s2l1w,
                   s2l1b.reshape(1, -1), s2l2w, s2l2b.reshape(1, -1))
    x2 = _glob(o2, s2g0w, s2g0b.reshape(1, -1), s2g1w, s2g1b.reshape(1, -1))

    pos2 = jnp.stack([p2x.reshape(128), p2y.reshape(128),
                      p2z.reshape(128)], axis=1)
    return (x2, pos2)


# speedup vs baseline: 3.8813x; 1.0066x over previous
import jax
import jax.numpy as jnp
import numpy as np
from jax.experimental import pallas as pl
from jax.experimental.pallas import tpu as pltpu

_PI = float(np.pi)


def _fps_body(px_ref, py_ref, pz_ref, ox_ref, oy_ref, oz_ref, *, m, s):
    X = px_ref[...]
    Y = py_ref[...]
    Z = pz_ref[...]
    sm = max(m // 128, 1)
    idxg = (jax.lax.broadcasted_iota(jnp.int32, (s, 128), 0) * 128
            + jax.lax.broadcasted_iota(jnp.int32, (s, 128), 1))
    idxm = (jax.lax.broadcasted_iota(jnp.int32, (sm, 128), 0) * 128
            + jax.lax.broadcasted_iota(jnp.int32, (sm, 128), 1))
    qx0 = px_ref[0, 0]
    qy0 = py_ref[0, 0]
    qz0 = pz_ref[0, 0]
    dx = X - qx0
    dy = Y - qy0
    dz = Z - qz0
    dists0 = (dx * dx + dy * dy) + dz * dz
    zf = jnp.zeros((sm, 128), jnp.float32)
    selx0 = jnp.where(idxm == 0, qx0, zf)
    sely0 = jnp.where(idxm == 0, qy0, zf)
    selz0 = jnp.where(idxm == 0, qz0, zf)

    def body(i, c):
        dists, qx, qy, qz, selx, sely, selz = c
        dx = X - qx
        dy = Y - qy
        dz = Z - qz
        d = (dx * dx + dy * dy) + dz * dz
        dists = jnp.minimum(dists, d)
        mx = jnp.max(dists)
        nxt = jnp.min(jnp.where(dists == mx, idxg, jnp.int32(2 ** 30)))
        mask = idxg == nxt
        nqx = jnp.sum(jnp.where(mask, X, 0.0))
        nqy = jnp.sum(jnp.where(mask, Y, 0.0))
        nqz = jnp.sum(jnp.where(mask, Z, 0.0))
        mi = idxm == i
        selx = jnp.where(mi, nqx, selx)
        sely = jnp.where(mi, nqy, sely)
        selz = jnp.where(mi, nqz, selz)
        return (dists, nqx, nqy, nqz, selx, sely, selz)

    c = jax.lax.fori_loop(1, m, body,
                          (dists0, qx0, qy0, qz0, selx0, sely0, selz0))
    ox_ref[...] = c[4]
    oy_ref[...] = c[5]
    oz_ref[...] = c[6]


def _fps(px, py, pz, m):
    s = px.shape[0]
    sm = max(m // 128, 1)
    import functools
    body = functools.partial(_fps_body, m=m, s=s)
    shp = jax.ShapeDtypeStruct((sm, 128), jnp.float32)
    return pl.pallas_call(body, out_shape=(shp, shp, shp))(px, py, pz)


def _bf(v):
    return v.astype(jnp.bfloat16).astype(jnp.float32)


def _qcoord(qref_vals, idxq, g):
    qmask = idxq == g
    return [jnp.sum(jnp.where(qmask, Q, 0.0)) for Q in qref_vals]


def _topk_rounds_multi(Cs, Xcs, Ycs, Zcs, pmats, qxs, qys, qzs):
    n = len(Cs)
    iota32 = jax.lax.broadcasted_iota(jnp.int32, (1, 32), 1)
    inf = jnp.float32(np.inf)
    pdxa = [jnp.zeros((1, 32), jnp.float32) for _ in range(n)]
    pdya = [jnp.zeros((1, 32), jnp.float32) for _ in range(n)]
    pdza = [jnp.zeros((1, 32), jnp.float32) for _ in range(n)]
    cola = [jnp.zeros((1, 32), jnp.int32) for _ in range(n)]
    Cs = list(Cs)
    ib = jnp.int32(2 ** 30)
    for j in range(32):
        ms = [jnp.min(Cs[q]) for q in range(n)]
        pms = [jnp.min(jnp.where(Cs[q] == ms[q], pmats[q], ib))
               for q in range(n)]
        ems = [pmats[q] == pms[q] for q in range(n)]
        jm = iota32 == j
        for q in range(n):
            em = ems[q]
            nx = jnp.sum(jnp.where(em, Xcs[q], 0.0))
            nyv = jnp.sum(jnp.where(em, Ycs[q], 0.0))
            nz = jnp.sum(jnp.where(em, Zcs[q], 0.0))
            pm = pms[q]
            pdxa[q] = jnp.where(jm, nx - qxs[q], pdxa[q])
            pdya[q] = jnp.where(jm, nyv - qys[q], pdya[q])
            pdza[q] = jnp.where(jm, nz - qzs[q], pdza[q])
            cola[q] = jnp.where(jm, pm, cola[q])
            Cs[q] = jnp.where(em, inf, Cs[q])
    return pdxa, pdya, pdza, cola


def _knn1_body(qx_ref, qy_ref, qz_ref, pxt_ref, pyt_ref, pzt_ref,
               pdx_ref, pdy_ref, pdz_ref, col_ref, d_ref, n2_ref,
               *, qper):
    step = pl.program_id(0)
    Xt = pxt_ref[...]
    Yt = pyt_ref[...]
    Zt = pzt_ref[...]

    @pl.when(step == 0)
    def _():
        n2_ref[...] = (Xt * Xt + Yt * Yt) + Zt * Zt

    Xtb, Ytb, Ztb = _bf(Xt), _bf(Yt), _bf(Zt)
    n2t = n2_ref[...]
    idxq = (jax.lax.broadcasted_iota(jnp.int32, (8, 128), 0) * 128
            + jax.lax.broadcasted_iota(jnp.int32, (8, 128), 1))
    QX = qx_ref[...]
    QY = qy_ref[...]
    QZ = qz_ref[...]
    lt = jnp.where(jax.lax.broadcasted_iota(jnp.int32, (128, 128), 0)
                   <= jax.lax.broadcasted_iota(jnp.int32, (128, 128), 1),
                   1.0, 0.0)
    iota64r = jax.lax.broadcasted_iota(jnp.int32, (64, 128), 0).astype(jnp.float32)
    iota_l128f = jax.lax.broadcasted_iota(jnp.int32, (1, 128), 1).astype(jnp.float32)
    iota_l64f = jax.lax.broadcasted_iota(jnp.int32, (1, 64), 1).astype(jnp.float32)
    iota_r128 = jax.lax.broadcasted_iota(jnp.int32, (128, 1), 0)
    hi_p = jax.lax.Precision.HIGHEST

    qxs, qys, qzs, rms, los, his = [], [], [], [], [], []
    for q in range(qper):
        g = step * qper + q
        qx, qy, qz = _qcoord((QX, QY, QZ), idxq, g)
        qxs.append(qx)
        qys.append(qy)
        qzs.append(qz)
        ny = (qx * qx + qy * qy) + qz * qz
        D = (ny + n2t) - 2.0 * ((_bf(qx) * Xtb + _bf(qy) * Ytb)
                                + _bf(qz) * Ztb)
        d_ref[q] = D
        rm = jnp.min(D, axis=0, keepdims=True)
        rms.append(rm)
        los.append(jnp.min(rm))
        his.append(jnp.max(rm))
    for _ in range(24):
        for q in range(qper):
            mid = 0.5 * (los[q] + his[q])
            cnt = jnp.sum(jnp.where(rms[q] <= mid, 1.0, 0.0))
            sel = cnt < 32.0
            los[q] = jnp.where(sel, mid, los[q])
            his[q] = jnp.where(sel, his[q], mid)
    Cs, Xcs, Ycs, Zcs, pmats = [], [], [], [], []
    cargs = (((1,), (1,)), ((), ()))
    for q in range(qper):
        mask = jnp.where(rms[q] <= his[q], 1.0, 0.0)
        cntf = jnp.sum(mask)
        rank = jax.lax.dot_general(mask, lt, (((1,), (0,)), ((), ())),
                                   preferred_element_type=jnp.float32)
        enc = jnp.where(mask > 0, rank - 1.0, jnp.float32(1e9))
        pt = jnp.where(iota64r == enc, 1.0, 0.0)
        Dv = d_ref[q][...]
        C = jax.lax.dot_general(Dv, pt, cargs, precision=hi_p,
                                preferred_element_type=jnp.float32)
        Xc = jax.lax.dot_general(Xt, pt, cargs, precision=hi_p,
                                 preferred_element_type=jnp.float32)
        Yc = jax.lax.dot_general(Yt, pt, cargs, precision=hi_p,
                                 preferred_element_type=jnp.float32)
        Zc = jax.lax.dot_general(Zt, pt, cargs, precision=hi_p,
                                 preferred_element_type=jnp.float32)
        cval = jax.lax.dot_general(iota_l128f, pt, cargs, precision=hi_p,
                                   preferred_element_type=jnp.float32)
        pmat = cval.astype(jnp.int32) * 128 + iota_r128
        C = jnp.where(iota_l64f < cntf, C, jnp.float32(np.inf))
        Cs.append(C)
        Xcs.append(Xc)
        Ycs.append(Yc)
        Zcs.append(Zc)
        pmats.append(pmat)
    pdxa, pdya, pdza, cola = _topk_rounds_multi(Cs, Xcs, Ycs, Zcs, pmats,
                                                qxs, qys, qzs)
    for q in range(qper):
        pdx_ref[0, q, :] = pdxa[q][0, :]
        pdy_ref[0, q, :] = pdya[q][0, :]
        pdz_ref[0, q, :] = pdza[q][0, :]
        col_ref[0, q, :] = cola[q][0, :]


def _knn1(qx, qy, qz, pxt, pyt, pzt, m, qper=4):
    import functools
    body = functools.partial(_knn1_body, qper=qper)
    shp = jax.ShapeDtypeStruct((m // qper, qper, 32), jnp.float32)
    shpi = jax.ShapeDtypeStruct((m // qper, qper, 32), jnp.int32)
    full_q = pl.BlockSpec((8, 128), lambda i: (0, 0))
    full_p = pl.BlockSpec((128, 128), lambda i: (0, 0))
    outb = pl.BlockSpec((1, qper, 32), lambda i: (i, 0, 0))
    return pl.pallas_call(
        body,
        grid=(m // qper,),
        in_specs=[full_q, full_q, full_q, full_p, full_p, full_p],
        out_specs=[outb, outb, outb, outb],
        out_shape=(shp, shp, shp, shpi),
        scratch_shapes=[pltpu.VMEM((qper, 128, 128), jnp.float32),
                        pltpu.VMEM((128, 128), jnp.float32)],
    )(qx, qy, qz, pxt, pyt, pzt)


def _knn2_body(qx_ref, qy_ref, qz_ref, px_ref, py_ref, pz_ref,
               pdx_ref, pdy_ref, pdz_ref, col_ref, n2_ref, *, qper):
    step = pl.program_id(0)
    X = px_ref[...]
    Y = py_ref[...]
    Z = pz_ref[...]

    @pl.when(step == 0)
    def _():
        n2_ref[...] = (X * X + Y * Y) + Z * Z

    Xb, Yb, Zb = _bf(X), _bf(Y), _bf(Z)
    n2 = n2_ref[...]
    idxq = jax.lax.broadcasted_iota(jnp.int32, (1, 128), 1)
    pmat = (jax.lax.broadcasted_iota(jnp.int32, (8, 128), 0) * 128
            + jax.lax.broadcasted_iota(jnp.int32, (8, 128), 1))
    QX = qx_ref[...]
    QY = qy_ref[...]
    QZ = qz_ref[...]

    qxs, qys, qzs, Ds = [], [], [], []
    for q in range(qper):
        g = step * qper + q
        qx, qy, qz = _qcoord((QX, QY, QZ), idxq, g)
        qxs.append(qx)
        qys.append(qy)
        qzs.append(qz)
        ny = (qx * qx + qy * qy) + qz * qz
        Ds.append((ny + n2) - 2.0 * ((_bf(qx) * Xb + _bf(qy) * Yb)
                                     + _bf(qz) * Zb))
    pdxa, pdya, pdza, cola = _topk_rounds_multi(
        Ds, [X] * qper, [Y] * qper, [Z] * qper, [pmat] * qper,
        qxs, qys, qzs)
    for q in range(qper):
        pdx_ref[0, q, :] = pdxa[q][0, :]
        pdy_ref[0, q, :] = pdya[q][0, :]
        pdz_ref[0, q, :] = pdza[q][0, :]
        col_ref[0, q, :] = cola[q][0, :]


def _knn2(qx, qy, qz, px, py, pz, m, qper=4):
    import functools
    body = functools.partial(_knn2_body, qper=qper)
    shp = jax.ShapeDtypeStruct((m // qper, qper, 32), jnp.float32)
    shpi = jax.ShapeDtypeStruct((m // qper, qper, 32), jnp.int32)
    full_q = pl.BlockSpec((1, 128), lambda i: (0, 0))
    full_p = pl.BlockSpec((8, 128), lambda i: (0, 0))
    outb = pl.BlockSpec((1, qper, 32), lambda i: (i, 0, 0))
    return pl.pallas_call(
        body,
        grid=(m // qper,),
        in_specs=[full_q, full_q, full_q, full_p, full_p, full_p],
        out_specs=[outb, outb, outb, outb],
        out_shape=(shp, shp, shp, shpi),
        scratch_shapes=[pltpu.VMEM((8, 128), jnp.float32)],
    )(qx, qy, qz, px, py, pz)


def _posenc_feat(pdx, pdy, pdz, n):
    e = jax.lax.broadcasted_iota(jnp.int32, (1, 64), 1)
    k = jnp.maximum(e - 3, 0)
    c = k // 20
    lf = (k % 20) // 2
    is_sin = (k % 2) == 0
    is_coord = e < 3
    coordid = jnp.where(is_coord, e, jnp.minimum(c, 2))
    freq = (jnp.int32(1) << lf).astype(jnp.float32) * _PI
    raw = jnp.where(coordid == 0, pdx,
                    jnp.where(coordid == 1, pdy, pdz))
    scaled = raw * jnp.where(is_coord, jnp.float32(1.0), freq)
    sv = jnp.sin(scaled)
    cv = jnp.cos(scaled)
    return jnp.where(is_coord, raw, jnp.where(is_sin, sv, cv))


def _sa1_body(pdx_ref, pdy_ref, pdz_ref, w0_ref, b0_ref, w1_ref, b1_ref,
              w2_ref, b2_ref, o1_ref):
    n = pdx_ref.shape[0]
    feat = _posenc_feat(pdx_ref[...], pdy_ref[...], pdz_ref[...], n)
    h = jax.lax.dot_general(feat, w0_ref[...], (((1,), (0,)), ((), ())),
                            preferred_element_type=jnp.float32) + b0_ref[...]
    h = jnp.maximum(h, 0.0)
    h = jax.lax.dot_general(h, w1_ref[...], (((1,), (0,)), ((), ())),
                            preferred_element_type=jnp.float32) + b1_ref[...]
    h = jnp.maximum(h, 0.0)
    h = jax.lax.dot_general(h, w2_ref[...], (((1,), (0,)), ((), ())),
                            preferred_element_type=jnp.float32) + b2_ref[...]
    hm = jnp.max(h.reshape(n // 32, 32, 128), axis=1)
    o1_ref[...] = hm


def _sa1_edge(pdxf, pdyf, pdzf, w0p, b0, w1, b1, w2, b2):
    ne = pdxf.shape[0]
    tile = 4096
    grid = ne // tile
    pdb = pl.BlockSpec((tile, 1), lambda i: (i, 0))
    wfull = lambda a: pl.BlockSpec(a.shape, lambda i: (0,) * a.ndim)
    return pl.pallas_call(
        _sa1_body,
        grid=(grid,),
        in_specs=[pdb, pdb, pdb, wfull(w0p), wfull(b0), wfull(w1),
                  wfull(b1), wfull(w2), wfull(b2)],
        out_specs=pl.BlockSpec((tile // 32, 128), lambda i: (i, 0)),
        out_shape=jax.ShapeDtypeStruct((ne // 32, 128), jnp.float32),
    )(pdxf, pdyf, pdzf, w0p, b0, w1, b1, w2, b2)


def _glob_body(x_ref, w0_ref, b0_ref, w1_ref, b1_ref, o_ref):
    h = jax.lax.dot_general(x_ref[...], w0_ref[...], (((1,), (0,)), ((), ())),
                            preferred_element_type=jnp.float32) + b0_ref[...]
    h = jnp.maximum(h, 0.0)
    h = jax.lax.dot_general(h, w1_ref[...], (((1,), (0,)), ((), ())),
                            preferred_element_type=jnp.float32) + b1_ref[...]
    o_ref[...] = h


def _glob(x, w0, b0, w1, b1):
    m = x.shape[0]
    return pl.pallas_call(
        _glob_body,
        out_shape=jax.ShapeDtypeStruct((m, w1.shape[1]), jnp.float32),
    )(x, w0, b0, w1, b1)


def _sa2_body(col_ref, pdx_ref, pdy_ref, pdz_ref, x1_ref,
              w0a_ref, w0b_ref, b0_ref, w1_ref, b1_ref, w2_ref, b2_ref,
              o2_ref):
    n = col_ref.shape[0]
    col = col_ref[...]
    og = jnp.zeros((n, 256), jnp.float32)
    for cb in range(8):
        iota_c = (jax.lax.broadcasted_iota(jnp.int32, (1, 128), 1)
                  + cb * 128)
        ohc = jnp.where(col == iota_c, 1.0, 0.0)
        og = og + jax.lax.dot_general(
            ohc, x1_ref[pl.ds(cb * 128, 128), :], (((1,), (0,)), ((), ())),
            preferred_element_type=jnp.float32)
    feat = _posenc_feat(pdx_ref[...], pdy_ref[...], pdz_ref[...], n)
    h = (jax.lax.dot_general(og, w0a_ref[...], (((1,), (0,)), ((), ())),
                             preferred_element_type=jnp.float32)
         + jax.lax.dot_general(feat, w0b_ref[...], (((1,), (0,)), ((), ())),
                               preferred_element_type=jnp.float32)
         + b0_ref[...])
    h = jnp.maximum(h, 0.0)
    h = jax.lax.dot_general(h, w1_ref[...], (((1,), (0,)), ((), ())),
                            preferred_element_type=jnp.float32) + b1_ref[...]
    h = jnp.maximum(h, 0.0)
    h = jax.lax.dot_general(h, w2_ref[...], (((1,), (0,)), ((), ())),
                            preferred_element_type=jnp.float32) + b2_ref[...]
    hm = jnp.max(h.reshape(n // 32, 32, 512), axis=1)
    o2_ref[...] = hm


def _sa2_edge(colf, pd2xf, pd2yf, pd2zf, x1, w0a, w0bp, b0, w1, b1, w2, b2):
    ne = colf.shape[0]
    tile = 1024
    grid = ne // tile
    cb = pl.BlockSpec((tile, 1), lambda i: (i, 0))
    wfull = lambda a: pl.BlockSpec(a.shape, lambda i: (0,) * a.ndim)
    return pl.pallas_call(
        _sa2_body,
        grid=(grid,),
        in_specs=[cb, cb, cb, cb, wfull(x1), wfull(w0a), wfull(w0bp),
                  wfull(b0), wfull(w1), wfull(b1), wfull(w2), wfull(b2)],
        out_specs=pl.BlockSpec((tile // 32, 512), lambda i: (i, 0)),
        out_shape=jax.ShapeDtypeStruct((ne // 32, 512), jnp.float32),
    )(colf, pd2xf, pd2yf, pd2zf, x1, w0a, w0bp, b0, w1, b1, w2, b2)


def kernel(pos, s1l0w, s1l0b, s1l1w, s1l1b, s1l2w, s1l2b,
           s1g0w, s1g0b, s1g1w, s1g1b,
           s2l0w, s2l0b, s2l1w, s2l1b, s2l2w, s2l2b,
           s2g0w, s2g0b, s2g1w, s2g1b):
    f = jnp.float32
    px = pos[:, 0].reshape(128, 128)
    py = pos[:, 1].reshape(128, 128)
    pz = pos[:, 2].reshape(128, 128)

    p1x, p1y, p1z = _fps(px, py, pz, 1024)
    pdx, pdy, pdz, _ = _knn1(p1x, p1y, p1z, px.T, py.T, pz.T, 1024, qper=16)
    w0p = jnp.pad(s1l0w, ((0, 1), (0, 0)))
    o1 = _sa1_edge(pdx.reshape(32768, 1), pdy.reshape(32768, 1),
                   pdz.reshape(32768, 1), w0p, s1l0b.reshape(1, -1),
                   s1l1w, s1l1b.reshape(1, -1), s1l2w, s1l2b.reshape(1, -1))
    x1 = _glob(o1, s1g0w, s1g0b.reshape(1, -1), s1g1w, s1g1b.reshape(1, -1))

    p2x, p2y, p2z = _fps(p1x, p1y, p1z, 128)
    pd2x, pd2y, pd2z, col2 = _knn2(p2x, p2y, p2z, p1x, p1y, p1z, 128, qper=8)
    w0a = s2l0w[:256]
    w0bp = jnp.pad(s2l0w[256:], ((0, 1), (0, 0)))
    o2 = _sa2_edge(col2.reshape(4096, 1), pd2x.reshape(4096, 1),
                   pd2y.reshape(4096, 1), pd2z.reshape(4096, 1), x1,
                   w0a, w0bp, s2l0b.reshape(1, -1), s2l1w,
                   s2l1b.reshape(1, -1), s2l2w, s2l2b.reshape(1, -1))
    x2 = _glob(o2, s2g0w, s2g0b.reshape(1, -1), s2g1w, s2g1b.reshape(1, -1))

    pos2 = jnp.stack([p2x.reshape(128), p2y.reshape(128),
                      p2z.reshape(128)], axis=1)
    return (x2, pos2)


# speedup vs baseline: 4.1072x; 1.0582x over previous
import jax
import jax.numpy as jnp
import numpy as np
from jax.experimental import pallas as pl
from jax.experimental.pallas import tpu as pltpu

_PI = float(np.pi)


def _fps_body(px_ref, py_ref, pz_ref, ox_ref, oy_ref, oz_ref, *, m, s):
    X = px_ref[...]
    Y = py_ref[...]
    Z = pz_ref[...]
    sm = max(m // 128, 1)
    idxg = (jax.lax.broadcasted_iota(jnp.int32, (s, 128), 0) * 128
            + jax.lax.broadcasted_iota(jnp.int32, (s, 128), 1))
    idxm = (jax.lax.broadcasted_iota(jnp.int32, (sm, 128), 0) * 128
            + jax.lax.broadcasted_iota(jnp.int32, (sm, 128), 1))
    qx0 = px_ref[0, 0]
    qy0 = py_ref[0, 0]
    qz0 = pz_ref[0, 0]
    dx = X - qx0
    dy = Y - qy0
    dz = Z - qz0
    dists0 = (dx * dx + dy * dy) + dz * dz
    zf = jnp.zeros((sm, 128), jnp.float32)
    selx0 = jnp.where(idxm == 0, qx0, zf)
    sely0 = jnp.where(idxm == 0, qy0, zf)
    selz0 = jnp.where(idxm == 0, qz0, zf)

    def body(i, c):
        dists, qx, qy, qz, selx, sely, selz = c
        dx = X - qx
        dy = Y - qy
        dz = Z - qz
        d = (dx * dx + dy * dy) + dz * dz
        dists = jnp.minimum(dists, d)
        mx = jnp.max(dists)
        nxt = jnp.min(jnp.where(dists == mx, idxg, jnp.int32(2 ** 30)))
        mask = idxg == nxt
        nqx = jnp.sum(jnp.where(mask, X, 0.0))
        nqy = jnp.sum(jnp.where(mask, Y, 0.0))
        nqz = jnp.sum(jnp.where(mask, Z, 0.0))
        mi = idxm == i
        selx = jnp.where(mi, nqx, selx)
        sely = jnp.where(mi, nqy, sely)
        selz = jnp.where(mi, nqz, selz)
        return (dists, nqx, nqy, nqz, selx, sely, selz)

    c = jax.lax.fori_loop(1, m, body,
                          (dists0, qx0, qy0, qz0, selx0, sely0, selz0))
    ox_ref[...] = c[4]
    oy_ref[...] = c[5]
    oz_ref[...] = c[6]


def _fps(px, py, pz, m):
    s = px.shape[0]
    sm = max(m // 128, 1)
    import functools
    body = functools.partial(_fps_body, m=m, s=s)
    shp = jax.ShapeDtypeStruct((sm, 128), jnp.float32)
    return pl.pallas_call(body, out_shape=(shp, shp, shp))(px, py, pz)


def _bf(v):
    return v.astype(jnp.bfloat16).astype(jnp.float32)


def _qcoord(qref_vals, idxq, g):
    qmask = idxq == g
    return [jnp.sum(jnp.where(qmask, Q, 0.0)) for Q in qref_vals]


def _topk_rounds_multi(Cs, Xcs, Ycs, Zcs, pmats, qxs, qys, qzs):
    n = len(Cs)
    iota32 = jax.lax.broadcasted_iota(jnp.int32, (1, 32), 1)
    inf = jnp.float32(np.inf)
    pdxa = [jnp.zeros((1, 32), jnp.float32) for _ in range(n)]
    pdya = [jnp.zeros((1, 32), jnp.float32) for _ in range(n)]
    pdza = [jnp.zeros((1, 32), jnp.float32) for _ in range(n)]
    cola = [jnp.zeros((1, 32), jnp.int32) for _ in range(n)]
    Cs = list(Cs)
    ib = jnp.int32(2 ** 30)
    for j in range(32):
        ms = [jnp.min(Cs[q]) for q in range(n)]
        pms = [jnp.min(jnp.where(Cs[q] == ms[q], pmats[q], ib))
               for q in range(n)]
        ems = [pmats[q] == pms[q] for q in range(n)]
        jm = iota32 == j
        for q in range(n):
            em = ems[q]
            nx = jnp.sum(jnp.where(em, Xcs[q], 0.0))
            nyv = jnp.sum(jnp.where(em, Ycs[q], 0.0))
            nz = jnp.sum(jnp.where(em, Zcs[q], 0.0))
            pm = pms[q]
            pdxa[q] = jnp.where(jm, nx - qxs[q], pdxa[q])
            pdya[q] = jnp.where(jm, nyv - qys[q], pdya[q])
            pdza[q] = jnp.where(jm, nz - qzs[q], pdza[q])
            cola[q] = jnp.where(jm, pm, cola[q])
            Cs[q] = jnp.where(em, inf, Cs[q])
    return pdxa, pdya, pdza, cola


def _knn1_body(qx_ref, qy_ref, qz_ref, pxt_ref, pyt_ref, pzt_ref,
               pdx_ref, pdy_ref, pdz_ref, col_ref, d_ref, n2_ref,
               *, qper):
    step = pl.program_id(0)
    Xt = pxt_ref[...]
    Yt = pyt_ref[...]
    Zt = pzt_ref[...]

    @pl.when(step == 0)
    def _():
        n2_ref[...] = (Xt * Xt + Yt * Yt) + Zt * Zt

    Xtb, Ytb, Ztb = _bf(Xt), _bf(Yt), _bf(Zt)
    n2t = n2_ref[...]
    idxq = (jax.lax.broadcasted_iota(jnp.int32, (8, 128), 0) * 128
            + jax.lax.broadcasted_iota(jnp.int32, (8, 128), 1))
    QX = qx_ref[...]
    QY = qy_ref[...]
    QZ = qz_ref[...]
    lt = jnp.where(jax.lax.broadcasted_iota(jnp.int32, (128, 128), 0)
                   <= jax.lax.broadcasted_iota(jnp.int32, (128, 128), 1),
                   1.0, 0.0)
    iota64r = jax.lax.broadcasted_iota(jnp.int32, (64, 128), 0).astype(jnp.float32)
    iota_l128f = jax.lax.broadcasted_iota(jnp.int32, (1, 128), 1).astype(jnp.float32)
    iota_l64f = jax.lax.broadcasted_iota(jnp.int32, (1, 64), 1).astype(jnp.float32)
    iota_r128 = jax.lax.broadcasted_iota(jnp.int32, (128, 1), 0)
    hi_p = jax.lax.Precision.HIGHEST

    qxs, qys, qzs, rms, los, his = [], [], [], [], [], []
    for q in range(qper):
        g = step * qper + q
        qx, qy, qz = _qcoord((QX, QY, QZ), idxq, g)
        qxs.append(qx)
        qys.append(qy)
        qzs.append(qz)
        ny = (qx * qx + qy * qy) + qz * qz
        D = (ny + n2t) - 2.0 * ((_bf(qx) * Xtb + _bf(qy) * Ytb)
                                + _bf(qz) * Ztb)
        d_ref[q] = D
        rm = jnp.min(D, axis=0, keepdims=True)
        rms.append(rm)
    RM = jnp.concatenate(rms, axis=0)
    lo = jnp.min(RM, axis=1, keepdims=True)
    hi = jnp.max(RM, axis=1, keepdims=True)
    for _ in range(24):
        mid = 0.5 * (lo + hi)
        cnt = jnp.sum(jnp.where(RM <= mid, 1.0, 0.0), axis=1, keepdims=True)
        sel = cnt < 32.0
        lo = jnp.where(sel, mid, lo)
        hi = jnp.where(sel, hi, mid)
    maskS = jnp.where(RM <= hi, 1.0, 0.0)
    cnts = jnp.sum(maskS, axis=1, keepdims=True)
    Cs, Xcs, Ycs, Zcs, pmats = [], [], [], [], []
    cargs = (((1,), (1,)), ((), ()))
    for q in range(qper):
        mask = maskS[q:q + 1, :]
        cntf = cnts[q:q + 1, :]
        rank = jax.lax.dot_general(mask, lt, (((1,), (0,)), ((), ())),
                                   preferred_element_type=jnp.float32)
        enc = jnp.where(mask > 0, rank - 1.0, jnp.float32(1e9))
        pt = jnp.where(iota64r == enc, 1.0, 0.0)
        Dv = d_ref[q][...]
        C = jax.lax.dot_general(Dv, pt, cargs, precision=hi_p,
                                preferred_element_type=jnp.float32)
        Xc = jax.lax.dot_general(Xt, pt, cargs, precision=hi_p,
                                 preferred_element_type=jnp.float32)
        Yc = jax.lax.dot_general(Yt, pt, cargs, precision=hi_p,
                                 preferred_element_type=jnp.float32)
        Zc = jax.lax.dot_general(Zt, pt, cargs, precision=hi_p,
                                 preferred_element_type=jnp.float32)
        cval = jax.lax.dot_general(iota_l128f, pt, cargs, precision=hi_p,
                                   preferred_element_type=jnp.float32)
        pmat = cval.astype(jnp.int32) * 128 + iota_r128
        C = jnp.where(iota_l64f < cntf, C, jnp.float32(np.inf))
        Cs.append(C)
        Xcs.append(Xc)
        Ycs.append(Yc)
        Zcs.append(Zc)
        pmats.append(pmat)
    pdxa, pdya, pdza, cola = _topk_rounds_multi(Cs, Xcs, Ycs, Zcs, pmats,
                                                qxs, qys, qzs)
    for q in range(qper):
        pdx_ref[0, q, :] = pdxa[q][0, :]
        pdy_ref[0, q, :] = pdya[q][0, :]
        pdz_ref[0, q, :] = pdza[q][0, :]
        col_ref[0, q, :] = cola[q][0, :]


def _knn1(qx, qy, qz, pxt, pyt, pzt, m, qper=4):
    import functools
    body = functools.partial(_knn1_body, qper=qper)
    shp = jax.ShapeDtypeStruct((m // qper, qper, 32), jnp.float32)
    shpi = jax.ShapeDtypeStruct((m // qper, qper, 32), jnp.int32)
    full_q = pl.BlockSpec((8, 128), lambda i: (0, 0))
    full_p = pl.BlockSpec((128, 128), lambda i: (0, 0))
    outb = pl.BlockSpec((1, qper, 32), lambda i: (i, 0, 0))
    return pl.pallas_call(
        body,
        grid=(m // qper,),
        in_specs=[full_q, full_q, full_q, full_p, full_p, full_p],
        out_specs=[outb, outb, outb, outb],
        out_shape=(shp, shp, shp, shpi),
        scratch_shapes=[pltpu.VMEM((qper, 128, 128), jnp.float32),
                        pltpu.VMEM((128, 128), jnp.float32)],
    )(qx, qy, qz, pxt, pyt, pzt)


def _knn2_body(qx_ref, qy_ref, qz_ref, px_ref, py_ref, pz_ref,
               pdx_ref, pdy_ref, pdz_ref, col_ref, n2_ref, *, qper):
    step = pl.program_id(0)
    X = px_ref[...]
    Y = py_ref[...]
    Z = pz_ref[...]

    @pl.when(step == 0)
    def _():
        n2_ref[...] = (X * X + Y * Y) + Z * Z

    Xb, Yb, Zb = _bf(X), _bf(Y), _bf(Z)
    n2 = n2_ref[...]
    idxq = jax.lax.broadcasted_iota(jnp.int32, (1, 128), 1)
    pmat = (jax.lax.broadcasted_iota(jnp.int32, (8, 128), 0) * 128
            + jax.lax.broadcasted_iota(jnp.int32, (8, 128), 1))
    QX = qx_ref[...]
    QY = qy_ref[...]
    QZ = qz_ref[...]

    qxs, qys, qzs, Ds = [], [], [], []
    for q in range(qper):
        g = step * qper + q
        qx, qy, qz = _qcoord((QX, QY, QZ), idxq, g)
        qxs.append(qx)
        qys.append(qy)
        qzs.append(qz)
        ny = (qx * qx + qy * qy) + qz * qz
        Ds.append((ny + n2) - 2.0 * ((_bf(qx) * Xb + _bf(qy) * Yb)
                                     + _bf(qz) * Zb))
    pdxa, pdya, pdza, cola = _topk_rounds_multi(
        Ds, [X] * qper, [Y] * qper, [Z] * qper, [pmat] * qper,
        qxs, qys, qzs)
    for q in range(qper):
        pdx_ref[0, q, :] = pdxa[q][0, :]
        pdy_ref[0, q, :] = pdya[q][0, :]
        pdz_ref[0, q, :] = pdza[q][0, :]
        col_ref[0, q, :] = cola[q][0, :]


def _knn2(qx, qy, qz, px, py, pz, m, qper=4):
    import functools
    body = functools.partial(_knn2_body, qper=qper)
    shp = jax.ShapeDtypeStruct((m // qper, qper, 32), jnp.float32)
    shpi = jax.ShapeDtypeStruct((m // qper, qper, 32), jnp.int32)
    full_q = pl.BlockSpec((1, 128), lambda i: (0, 0))
    full_p = pl.BlockSpec((8, 128), lambda i: (0, 0))
    outb = pl.BlockSpec((1, qper, 32), lambda i: (i, 0, 0))
    return pl.pallas_call(
        body,
        grid=(m // qper,),
        in_specs=[full_q, full_q, full_q, full_p, full_p, full_p],
        out_specs=[outb, outb, outb, outb],
        out_shape=(shp, shp, shp, shpi),
        scratch_shapes=[pltpu.VMEM((8, 128), jnp.float32)],
    )(qx, qy, qz, px, py, pz)


def _posenc_feat(pdx, pdy, pdz, n):
    e = jax.lax.broadcasted_iota(jnp.int32, (1, 64), 1)
    k = jnp.maximum(e - 3, 0)
    c = k // 20
    lf = (k % 20) // 2
    is_sin = (k % 2) == 0
    is_coord = e < 3
    coordid = jnp.where(is_coord, e, jnp.minimum(c, 2))
    freq = (jnp.int32(1) << lf).astype(jnp.float32) * _PI
    raw = jnp.where(coordid == 0, pdx,
                    jnp.where(coordid == 1, pdy, pdz))
    scaled = raw * jnp.where(is_coord, jnp.float32(1.0), freq)
    sv = jnp.sin(scaled)
    cv = jnp.cos(scaled)
    return jnp.where(is_coord, raw, jnp.where(is_sin, sv, cv))


def _sa1_body(pdx_ref, pdy_ref, pdz_ref, w0_ref, b0_ref, w1_ref, b1_ref,
              w2_ref, b2_ref, o1_ref):
    n = pdx_ref.shape[0]
    feat = _posenc_feat(pdx_ref[...], pdy_ref[...], pdz_ref[...], n)
    h = jax.lax.dot_general(feat, w0_ref[...], (((1,), (0,)), ((), ())),
                            preferred_element_type=jnp.float32) + b0_ref[...]
    h = jnp.maximum(h, 0.0)
    h = jax.lax.dot_general(h, w1_ref[...], (((1,), (0,)), ((), ())),
                            preferred_element_type=jnp.float32) + b1_ref[...]
    h = jnp.maximum(h, 0.0)
    h = jax.lax.dot_general(h, w2_ref[...], (((1,), (0,)), ((), ())),
                            preferred_element_type=jnp.float32) + b2_ref[...]
    hm = jnp.max(h.reshape(n // 32, 32, 128), axis=1)
    o1_ref[...] = hm


def _sa1_edge(pdxf, pdyf, pdzf, w0p, b0, w1, b1, w2, b2):
    ne = pdxf.shape[0]
    tile = 4096
    grid = ne // tile
    pdb = pl.BlockSpec((tile, 1), lambda i: (i, 0))
    wfull = lambda a: pl.BlockSpec(a.shape, lambda i: (0,) * a.ndim)
    return pl.pallas_call(
        _sa1_body,
        grid=(grid,),
        in_specs=[pdb, pdb, pdb, wfull(w0p), wfull(b0), wfull(w1),
                  wfull(b1), wfull(w2), wfull(b2)],
        out_specs=pl.BlockSpec((tile // 32, 128), lambda i: (i, 0)),
        out_shape=jax.ShapeDtypeStruct((ne // 32, 128), jnp.float32),
    )(pdxf, pdyf, pdzf, w0p, b0, w1, b1, w2, b2)


def _glob_body(x_ref, w0_ref, b0_ref, w1_ref, b1_ref, o_ref):
    h = jax.lax.dot_general(x_ref[...], w0_ref[...], (((1,), (0,)), ((), ())),
                            preferred_element_type=jnp.float32) + b0_ref[...]
    h = jnp.maximum(h, 0.0)
    h = jax.lax.dot_general(h, w1_ref[...], (((1,), (0,)), ((), ())),
                            preferred_element_type=jnp.float32) + b1_ref[...]
    o_ref[...] = h


def _glob(x, w0, b0, w1, b1):
    m = x.shape[0]
    return pl.pallas_call(
        _glob_body,
        out_shape=jax.ShapeDtypeStruct((m, w1.shape[1]), jnp.float32),
    )(x, w0, b0, w1, b1)


def _sa2_body(col_ref, pdx_ref, pdy_ref, pdz_ref, x1_ref,
              w0a_ref, w0b_ref, b0_ref, w1_ref, b1_ref, w2_ref, b2_ref,
              o2_ref):
    n = col_ref.shape[0]
    col = col_ref[...]
    og = jnp.zeros((n, 256), jnp.float32)
    for cb in range(8):
        iota_c = (jax.lax.broadcasted_iota(jnp.int32, (1, 128), 1)
                  + cb * 128)
        ohc = jnp.where(col == iota_c, 1.0, 0.0)
        og = og + jax.lax.dot_general(
            ohc, x1_ref[pl.ds(cb * 128, 128), :], (((1,), (0,)), ((), ())),
            preferred_element_type=jnp.float32)
    feat = _posenc_feat(pdx_ref[...], pdy_ref[...], pdz_ref[...], n)
    h = (jax.lax.dot_general(og, w0a_ref[...], (((1,), (0,)), ((), ())),
                             preferred_element_type=jnp.float32)
         + jax.lax.dot_general(feat, w0b_ref[...], (((1,), (0,)), ((), ())),
                               preferred_element_type=jnp.float32)
         + b0_ref[...])
    h = jnp.maximum(h, 0.0)
    h = jax.lax.dot_general(h, w1_ref[...], (((1,), (0,)), ((), ())),
                            preferred_element_type=jnp.float32) + b1_ref[...]
    h = jnp.maximum(h, 0.0)
    h = jax.lax.dot_general(h, w2_ref[...], (((1,), (0,)), ((), ())),
                            preferred_element_type=jnp.float32) + b2_ref[...]
    hm = jnp.max(h.reshape(n // 32, 32, 512), axis=1)
    o2_ref[...] = hm


def _sa2_edge(colf, pd2xf, pd2yf, pd2zf, x1, w0a, w0bp, b0, w1, b1, w2, b2):
    ne = colf.shape[0]
    tile = 1024
    grid = ne // tile
    cb = pl.BlockSpec((tile, 1), lambda i: (i, 0))
    wfull = lambda a: pl.BlockSpec(a.shape, lambda i: (0,) * a.ndim)
    return pl.pallas_call(
        _sa2_body,
        grid=(grid,),
        in_specs=[cb, cb, cb, cb, wfull(x1), wfull(w0a), wfull(w0bp),
                  wfull(b0), wfull(w1), wfull(b1), wfull(w2), wfull(b2)],
        out_specs=pl.BlockSpec((tile // 32, 512), lambda i: (i, 0)),
        out_shape=jax.ShapeDtypeStruct((ne // 32, 512), jnp.float32),
    )(colf, pd2xf, pd2yf, pd2zf, x1, w0a, w0bp, b0, w1, b1, w2, b2)


def kernel(pos, s1l0w, s1l0b, s1l1w, s1l1b, s1l2w, s1l2b,
           s1g0w, s1g0b, s1g1w, s1g1b,
           s2l0w, s2l0b, s2l1w, s2l1b, s2l2w, s2l2b,
           s2g0w, s2g0b, s2g1w, s2g1b):
    f = jnp.float32
    px = pos[:, 0].reshape(128, 128)
    py = pos[:, 1].reshape(128, 128)
    pz = pos[:, 2].reshape(128, 128)

    p1x, p1y, p1z = _fps(px, py, pz, 1024)
    pdx, pdy, pdz, _ = _knn1(p1x, p1y, p1z, px.T, py.T, pz.T, 1024, qper=16)
    w0p = jnp.pad(s1l0w, ((0, 1), (0, 0)))
    o1 = _sa1_edge(pdx.reshape(32768, 1), pdy.reshape(32768, 1),
                   pdz.reshape(32768, 1), w0p, s1l0b.reshape(1, -1),
                   s1l1w, s1l1b.reshape(1, -1), s1l2w, s1l2b.reshape(1, -1))
    x1 = _glob(o1, s1g0w, s1g0b.reshape(1, -1), s1g1w, s1g1b.reshape(1, -1))

    p2x, p2y, p2z = _fps(p1x, p1y, p1z, 128)
    pd2x, pd2y, pd2z, col2 = _knn2(p2x, p2y, p2z, p1x, p1y, p1z, 128, qper=8)
    w0a = s2l0w[:256]
    w0bp = jnp.pad(s2l0w[256:], ((0, 1), (0, 0)))
    o2 = _sa2_edge(col2.reshape(4096, 1), pd2x.reshape(4096, 1),
                   pd2y.reshape(4096, 1), pd2z.reshape(4096, 1), x1,
                   w0a, w0bp, s2l0b.reshape(1, -1), s2l1w,
                   s2l1b.reshape(1, -1), s2l2w, s2l2b.reshape(1, -1))
    x2 = _glob(o2, s2g0w, s2g0b.reshape(1, -1), s2g1w, s2g1b.reshape(1, -1))

    pos2 = jnp.stack([p2x.reshape(128), p2y.reshape(128),
                      p2z.reshape(128)], axis=1)
    return (x2, pos2)
